# speedup vs baseline: 6.1918x; 6.1918x over previous
import functools

import jax
import jax.numpy as jnp
from jax.experimental import pallas as pl


def _epilogue_body(num_ref, den_ref, bias_ref, scale_ref, shift_ref, o_ref, *, H, C):
    num = num_ref[...]
    den = den_ref[...]
    denf = jnp.repeat(den, C, axis=1)
    h = num / (denf + 1e-16) + bias_ref[...]
    o_ref[...] = jnp.maximum(h * scale_ref[...] + shift_ref[...], 0.0)


def _epilogue(num, den, bias, bn_g, bn_b, bn_rm, bn_rv, H, C):
    n = num.shape[0]
    HC = H * C
    inv = bn_g / jnp.sqrt(bn_rv + 1e-5)
    shift = bn_b - bn_rm * inv
    BLK = 1024
    grid = (pl.cdiv(n, BLK),)
    return pl.pallas_call(
        functools.partial(_epilogue_body, H=H, C=C),
        grid=grid,
        in_specs=[
            pl.BlockSpec((BLK, HC), lambda i: (i, 0)),
            pl.BlockSpec((BLK, H), lambda i: (i, 0)),
            pl.BlockSpec((1, HC), lambda i: (0, 0)),
            pl.BlockSpec((1, HC), lambda i: (0, 0)),
            pl.BlockSpec((1, HC), lambda i: (0, 0)),
        ],
        out_specs=pl.BlockSpec((BLK, HC), lambda i: (i, 0)),
        out_shape=jax.ShapeDtypeStruct((n, HC), jnp.float32),
    )(num, den, bias.reshape(1, HC), inv.reshape(1, HC), shift.reshape(1, HC))


def _gat_fused(x, src, dst, Wl, Wr, att, H, C):
    n = x.shape[0]
    xl = (x @ Wl).reshape(n, H, C)
    xr = (x @ Wr).reshape(n, H, C)
    e = jax.nn.leaky_relu(xl[src] + xr[dst], negative_slope=0.2)
    logits = jnp.einsum('ehc,hc->eh', e, att)
    p = jnp.exp(logits)
    num = jax.ops.segment_sum((p[:, :, None] * xl[src]).reshape(-1, H * C), dst,
                              num_segments=n)
    den = jax.ops.segment_sum(p, dst, num_segments=n)
    return num, den


def kernel(x, edge_index, batch, global_feat, Wl1, Wr1, att1, bias1, bn1_g,
           bn1_b, bn1_rm, bn1_rv, Wl2, Wr2, att2, bias2, bn2_g, bn2_b,
           bn2_rm, bn2_rv, fc1_w, fc1_b, fc2_w, fc2_b):
    H1, C1 = att1.shape
    H2, C2 = att2.shape
    G = global_feat.shape[0]
    src, dst = edge_index[0], edge_index[1]

    num1, den1 = _gat_fused(x, src, dst, Wl1, Wr1, att1, H1, C1)
    h = _epilogue(num1, den1, bias1, bn1_g, bn1_b, bn1_rm, bn1_rv, H1, C1)
    num2, den2 = _gat_fused(h, src, dst, Wl2, Wr2, att2, H2, C2)
    h = _epilogue(num2, den2, bias2, bn2_g, bn2_b, bn2_rm, bn2_rv, H2, C2)

    sums = jax.ops.segment_sum(h, batch, num_segments=G)
    cnt = jax.ops.segment_sum(jnp.ones((h.shape[0],), dtype=h.dtype), batch,
                              num_segments=G)
    pooled = sums / jnp.maximum(cnt, 1.0)[:, None]
    z = jnp.concatenate([pooled, global_feat], axis=1)
    z = jax.nn.relu(z @ fc1_w + fc1_b)
    out = (z @ fc2_w + fc2_b).squeeze()
    return out


# speedup vs baseline: 11.0450x; 1.7838x over previous
import functools

import numpy as np
import jax
import jax.numpy as jnp
from jax import lax
from jax.experimental import pallas as pl
from jax.experimental.pallas import tpu as pltpu
from jax.experimental.pallas import tpu_sc as plsc

NW = 32
LBW = 7
W_BKT = 1 << LBW
BB = 2048
EBLK = 32
RANK_BASE = 1


def _roundup(x, m):
    return (x + m - 1) // m * m


def _sc_mesh():
    return plsc.VectorSubcoreMesh(core_axis_name="c", subcore_axis_name="s")


_SC_PARAMS = pltpu.CompilerParams(needs_layout_passes=False)


def _perm_idx(H, C):
    HC = H * C
    k = np.arange(HC)
    j, l = k // 16, k % 16
    h = l % H
    c = j * (16 // H) + l // H
    return (h * C + c).astype(np.int32)


def _bin_hist(dst_pad, E, NB, EPW):
    nblk = EPW // BB

    @functools.partial(
        pl.kernel, mesh=_sc_mesh(), compiler_params=_SC_PARAMS,
        out_type=jax.ShapeDtypeStruct((NW, NB), jnp.int32),
        scratch_types=[
            pltpu.VMEM((BB,), jnp.int32),
            pltpu.VMEM((NB,), jnp.int32),
        ],
    )
    def k(dst_hbm, hist_hbm, buf_v, hist_v):
        wid = lax.axis_index("s") * 2 + lax.axis_index("c")
        wstart = wid * EPW
        zero16 = jnp.zeros((16,), jnp.int32)

        def zbody(i, c):
            hist_v[pl.ds(i * 16, 16)] = zero16
            return c
        lax.fori_loop(0, NB // 16, zbody, 0, unroll=True)

        iota = lax.iota(jnp.int32, 16)

        def blk(kb, c):
            pltpu.sync_copy(dst_hbm.at[pl.ds(wstart + kb * BB, BB)], buf_v)

            def chunk(i, c2):
                d = buf_v[pl.ds(i * 16, 16)]
                gidx = wstart + kb * BB + i * 16 + iota
                b = jnp.where(gidx < E, d >> LBW, NB - 1)
                rank, lastm = plsc.scan_count(b)
                plsc.addupdate_scatter(
                    hist_v, [b], rank + (1 - RANK_BASE), mask=lastm)
                return c2
            return lax.fori_loop(0, BB // 16, chunk, c)
        lax.fori_loop(0, nblk, blk, 0)
        pltpu.sync_copy(hist_v, hist_hbm.at[wid])
    return k(dst_pad)


def _bin_place(src_pad, dst_pad, hist, E, NB, EPW, EPAD):
    nblk = EPW // BB
    NB2 = NB + 16

    @functools.partial(
        pl.kernel, mesh=_sc_mesh(), compiler_params=_SC_PARAMS,
        out_type=(
            jax.ShapeDtypeStruct((EPAD,), jnp.int32),
            jax.ShapeDtypeStruct((EPAD,), jnp.int32),
            jax.ShapeDtypeStruct((NB2,), jnp.int32),
            jax.ShapeDtypeStruct((NB2,), jnp.int32),
        ),
        scratch_types=[
            pltpu.VMEM((NW, NB), jnp.int32),
            pltpu.VMEM((NB2,), jnp.int32),
            pltpu.VMEM((NB2,), jnp.int32),
            pltpu.VMEM((NB2,), jnp.int32),
            pltpu.VMEM((BB,), jnp.int32),
            pltpu.VMEM((BB,), jnp.int32),
            pltpu.VMEM((BB,), jnp.int32),
            pltpu.SemaphoreType.DMA,
            pltpu.SemaphoreType.DMA,
        ],
    )
    def k(src_hbm, dst_hbm, hist_hbm, esrc_hbm, edst_hbm, pstart_hbm,
          cnt_hbm, histl_v, off_v, pst_v, tot_v, sbuf_v, dbuf_v, pos_v,
          sem1, sem2):
        wid = lax.axis_index("s") * 2 + lax.axis_index("c")
        wstart = wid * EPW
        pltpu.sync_copy(hist_hbm, histl_v)
        zero16 = jnp.zeros((16,), jnp.int32)
        for cz in range(NB2 // 16):
            off_v[pl.ds(cz * 16, 16)] = zero16
            pst_v[pl.ds(cz * 16, 16)] = zero16
            tot_v[pl.ds(cz * 16, 16)] = zero16

        carry = jnp.int32(0)
        for c in range(NB // 16):
            sl = pl.ds(c * 16, 16)

            def addrow(wp, acc, _sl=sl):
                return acc + histl_v[wp, _sl]
            tot = lax.fori_loop(0, NW, addrow, zero16)
            mine = lax.fori_loop(0, wid, addrow, zero16)
            padded = jnp.bitwise_and(tot + 15, jnp.int32(-16))
            cums = plsc.cumsum(padded)
            pstart_c = cums - padded + carry
            off_v[sl] = pstart_c + mine
            pst_v[sl] = pstart_c
            tot_v[sl] = tot
            carry = carry + cums[15]

        @pl.when(wid == 0)
        def _():
            pltpu.sync_copy(pst_v, pstart_hbm)
            pltpu.sync_copy(tot_v, cnt_hbm)

        iota = lax.iota(jnp.int32, 16)

        def blk(kb, c):
            base = wstart + kb * BB
            pltpu.sync_copy(src_hbm.at[pl.ds(base, BB)], sbuf_v)
            pltpu.sync_copy(dst_hbm.at[pl.ds(base, BB)], dbuf_v)

            def chunk(i, c2):
                d = dbuf_v[pl.ds(i * 16, 16)]
                gidx = base + i * 16 + iota
                b = jnp.where(gidx < E, d >> LBW, NB - 1)
                bs = plsc.load_gather(off_v, [b])
                rank, lastm = plsc.scan_count(b)
                pos = jnp.minimum(bs + (rank - RANK_BASE), EPAD - 1)
                plsc.store_scatter(off_v, [b], pos + 1, mask=lastm)
                pos_v[pl.ds(i * 16, 16)] = pos
                return c2
            lax.fori_loop(0, BB // 16, chunk, c)
            cp1 = pltpu.async_copy(sbuf_v, esrc_hbm.at[pos_v], sem1)
            cp2 = pltpu.async_copy(dbuf_v, edst_hbm.at[pos_v], sem2)
            cp1.wait()
            cp2.wait()
            return c
        lax.fori_loop(0, nblk, blk, 0)
    return k(src_pad, dst_pad, hist)


def _gat_edge(xl, xr, att_p, esrc, edst, pstart, cnt, H, C, B, NB2, NLIM):
    HC = H * C
    NPAD = B * W_BKT
    NCH = HC // 16
    nbk_max = _roundup(B, NW) // NW

    @functools.partial(
        pl.kernel, mesh=_sc_mesh(), compiler_params=_SC_PARAMS,
        out_type=(
            jax.ShapeDtypeStruct((NPAD * HC,), jnp.float32),
            jax.ShapeDtypeStruct((NPAD * 16,), jnp.float32),
        ),
        scratch_types=[
            pltpu.VMEM((W_BKT * HC,), jnp.float32),
            pltpu.VMEM((W_BKT * 16,), jnp.float32),
            pltpu.VMEM((EBLK, HC), jnp.float32),
            pltpu.VMEM((EBLK, HC), jnp.float32),
            pltpu.VMEM((HC,), jnp.float32),
            pltpu.VMEM((EBLK,), jnp.int32),
            pltpu.VMEM((EBLK,), jnp.int32),
            pltpu.VMEM((EBLK + 16,), jnp.int32),
            pltpu.VMEM((32,), jnp.float32),
            pltpu.VMEM((NB2,), jnp.int32),
            pltpu.VMEM((NB2,), jnp.int32),
            pltpu.SemaphoreType.DMA,
            pltpu.SemaphoreType.DMA,
        ],
    )
    def k(xl_hbm, xr_hbm, att_hbm, esrc_hbm, edst_hbm, pstart_hbm, cnt_hbm,
          num_hbm, den_hbm, acc_v, den_v, xlb_v, xrb_v, att_v, sidx_v,
          dgidx_v, didx_v, rot_v, pst_v, cnt_v, sem1, sem2):
        wid = lax.axis_index("s") * 2 + lax.axis_index("c")
        pltpu.sync_copy(att_hbm, att_v)
        pltpu.sync_copy(pstart_hbm, pst_v)
        pltpu.sync_copy(cnt_hbm, cnt_v)
        zf16 = jnp.zeros((16,), jnp.float32)

        def bucket_body(t, c):
            b = wid + t * NW

            @pl.when(b < B)
            def _():
                bstart = pl.multiple_of(pst_v[pl.ds(b, 16)][0], 16)
                bcnt = cnt_v[pl.ds(b, 16)][0]

                def z1(i, c2):
                    acc_v[pl.ds(i * 16, 16)] = zf16
                    return c2
                lax.fori_loop(0, W_BKT * NCH, z1, 0)

                def z2(i, c2):
                    den_v[pl.ds(i * 16, 16)] = zf16
                    return c2
                lax.fori_loop(0, W_BKT, z2, 0)

                def blk(kb, c2):
                    eb = bstart + kb * EBLK
                    pltpu.sync_copy(esrc_hbm.at[pl.ds(eb, EBLK)], sidx_v)
                    pltpu.sync_copy(edst_hbm.at[pl.ds(eb, EBLK)],
                                    didx_v.at[pl.ds(0, EBLK)])
                    for q in range(EBLK // 16):
                        sq = pl.ds(q * 16, 16)
                        sidx_v[sq] = jnp.clip(sidx_v[sq], 0, NLIM - 1)
                        dgidx_v[sq] = jnp.clip(didx_v[sq], 0, NLIM - 1)
                    g1 = pltpu.async_copy(xl_hbm.at[sidx_v], xlb_v, sem1)
                    g2 = pltpu.async_copy(xr_hbm.at[dgidx_v], xrb_v, sem2)
                    g1.wait()
                    g2.wait()
                    nleft = jnp.minimum(bcnt - kb * EBLK, EBLK)

                    def edge(e, c3):
                        dstoff = didx_v[pl.ds(e, 16)][0] - b * W_BKT
                        a0 = zf16
                        a1 = zf16
                        for j in range(NCH):
                            sl = pl.ds(j * 16, 16)
                            s = xlb_v[e, sl] + xrb_v[e, sl]
                            tlr = jnp.maximum(s, 0.2 * s)
                            if j % 2 == 0:
                                a0 = a0 + tlr * att_v[sl]
                            else:
                                a1 = a1 + tlr * att_v[sl]
                        acc = a0 + a1
                        step = H
                        while step < 16:
                            rot_v[pl.ds(0, 16)] = acc
                            rot_v[pl.ds(16, 16)] = acc
                            acc = acc + rot_v[pl.ds(step, 16)]
                            step *= 2
                        p = jnp.exp(acc)
                        den_v[pl.ds(dstoff * 16, 16)] += p
                        abase = dstoff * HC
                        for j in range(NCH):
                            sl = pl.ds(j * 16, 16)
                            acc_v[pl.ds(abase + j * 16, 16)] += p * xlb_v[e, sl]
                        return c3
                    lax.fori_loop(0, nleft, edge, 0)
                    return c2
                lax.fori_loop(0, pl.cdiv(bcnt, EBLK), blk, 0)
                pltpu.sync_copy(acc_v,
                                num_hbm.at[pl.ds(b * W_BKT * HC, W_BKT * HC)])
                pltpu.sync_copy(den_v,
                                den_hbm.at[pl.ds(b * W_BKT * 16, W_BKT * 16)])
            return c
        lax.fori_loop(0, nbk_max, bucket_body, 0)
    return k(xl, xr, att_p, esrc, edst, pstart, cnt)


def _mm_pair_body(x_ref, wl_ref, wr_ref, ol_ref, or_ref):
    xb = x_ref[...]
    ol_ref[...] = jnp.dot(xb, wl_ref[...], preferred_element_type=jnp.float32)
    or_ref[...] = jnp.dot(xb, wr_ref[...], preferred_element_type=jnp.float32)


def _mm_pair(x, wl, wr):
    M, K = x.shape
    HC = wl.shape[1]
    BLK = 1024
    grid = (pl.cdiv(M, BLK),)
    return pl.pallas_call(
        _mm_pair_body,
        grid=grid,
        in_specs=[
            pl.BlockSpec((BLK, K), lambda i: (i, 0)),
            pl.BlockSpec((K, HC), lambda i: (0, 0)),
            pl.BlockSpec((K, HC), lambda i: (0, 0)),
        ],
        out_specs=[
            pl.BlockSpec((BLK, HC), lambda i: (i, 0)),
            pl.BlockSpec((BLK, HC), lambda i: (i, 0)),
        ],
        out_shape=[
            jax.ShapeDtypeStruct((M, HC), jnp.float32),
            jax.ShapeDtypeStruct((M, HC), jnp.float32),
        ],
    )(x, wl, wr)


def _epilogue_body(num_ref, den_ref, bias_ref, scale_ref, shift_ref, o_ref,
                   *, reps):
    num = num_ref[...]
    den = jnp.concatenate([den_ref[...]] * reps, axis=1)
    h = num / (den + 1e-16) + bias_ref[...]
    o_ref[...] = jnp.maximum(h * scale_ref[...] + shift_ref[...], 0.0)


def _epilogue(num, den, bias, bn_g, bn_b, bn_rm, bn_rv):
    n, HC = num.shape
    inv = bn_g / jnp.sqrt(bn_rv + 1e-5)
    shift = bn_b - bn_rm * inv
    BLK = 1024
    return pl.pallas_call(
        functools.partial(_epilogue_body, reps=HC // 16),
        grid=(pl.cdiv(n, BLK),),
        in_specs=[
            pl.BlockSpec((BLK, HC), lambda i: (i, 0)),
            pl.BlockSpec((BLK, 16), lambda i: (i, 0)),
            pl.BlockSpec((1, HC), lambda i: (0, 0)),
            pl.BlockSpec((1, HC), lambda i: (0, 0)),
            pl.BlockSpec((1, HC), lambda i: (0, 0)),
        ],
        out_specs=pl.BlockSpec((BLK, HC), lambda i: (i, 0)),
        out_shape=jax.ShapeDtypeStruct((n, HC), jnp.float32),
    )(num, den, bias.reshape(1, HC), inv.reshape(1, HC), shift.reshape(1, HC))


def _pool_body(b_ref, h_ref, sum_ref, cnt_ref, *, G):
    i = pl.program_id(0)
    bb = b_ref[0, 0, :]
    onehot = (lax.broadcasted_iota(jnp.int32, (G, bb.shape[0]), 0)
              == bb[None, :]).astype(jnp.float32)

    @pl.when(i == 0)
    def _():
        sum_ref[...] = jnp.zeros_like(sum_ref)
        cnt_ref[...] = jnp.zeros_like(cnt_ref)

    sum_ref[...] += jnp.dot(onehot, h_ref[...],
                            preferred_element_type=jnp.float32)
    cnt_ref[...] += jnp.dot(onehot,
                            jnp.ones((bb.shape[0], 128), jnp.float32),
                            preferred_element_type=jnp.float32)


def _pool(h, batch3, G):
    NP, HC = h.shape
    BLK = 256
    nblk = NP // BLK
    return pl.pallas_call(
        functools.partial(_pool_body, G=G),
        grid=(nblk,),
        in_specs=[
            pl.BlockSpec((1, 1, BLK), lambda i: (i, 0, 0)),
            pl.BlockSpec((BLK, HC), lambda i: (i, 0)),
        ],
        out_specs=[
            pl.BlockSpec((G, HC), lambda i: (0, 0)),
            pl.BlockSpec((G, 128), lambda i: (0, 0)),
        ],
        out_shape=[
            jax.ShapeDtypeStruct((G, HC), jnp.float32),
            jax.ShapeDtypeStruct((G, 128), jnp.float32),
        ],
    )(batch3, h)


def _head_body(sum_ref, cnt_ref, glob_ref, w1a_ref, w1b_ref, b1_ref,
               w2_ref, b2_ref, o_ref):
    cnt = jnp.maximum(cnt_ref[:, 0:1], 1.0)
    pooled = sum_ref[...] / cnt
    z = (jnp.dot(pooled, w1a_ref[...], preferred_element_type=jnp.float32)
         + jnp.dot(glob_ref[...], w1b_ref[...],
                   preferred_element_type=jnp.float32)
         + b1_ref[...])
    z = jnp.maximum(z, 0.0)
    o_ref[...] = jnp.dot(z, w2_ref[...],
                         preferred_element_type=jnp.float32) + b2_ref[...]


def _head(sums, cnts, globp, w1a, w1b, b1, w2, b2):
    G = sums.shape[0]
    return pl.pallas_call(
        _head_body,
        out_shape=jax.ShapeDtypeStruct((G, 8), jnp.float32),
    )(sums, cnts, globp, w1a, w1b, b1.reshape(1, -1), w2, b2.reshape(1, -1))


def kernel(x, edge_index, batch, global_feat, Wl1, Wr1, att1, bias1, bn1_g,
           bn1_b, bn1_rm, bn1_rv, Wl2, Wr2, att2, bias2, bn2_g, bn2_b,
           bn2_rm, bn2_rv, fc1_w, fc1_b, fc2_w, fc2_b):
    N = x.shape[0]
    E = edge_index.shape[1]
    G = global_feat.shape[0]
    H1, C1 = att1.shape
    H2, C2 = att2.shape
    HC1, HC2 = H1 * C1, H2 * C2

    B = -(-N // W_BKT)
    NB = _roundup(B + 1, 16)
    NB2 = NB + 16
    NPAD = B * W_BKT
    EPW = _roundup(-(-E // NW), BB)
    EPAD = NW * EPW + 16 * NB

    p1 = jnp.asarray(_perm_idx(H1, C1))
    p2 = jnp.asarray(_perm_idx(H2, C2))

    Wl1p, Wr1p = Wl1[:, p1], Wr1[:, p1]
    att1p = att1.reshape(HC1)[p1]
    bias1p, g1p, b1p = bias1[p1], bn1_g[p1], bn1_b[p1]
    rm1p, rv1p = bn1_rm[p1], bn1_rv[p1]
    Wl2p, Wr2p = Wl2[p1][:, p2], Wr2[p1][:, p2]
    att2p = att2.reshape(HC2)[p2]
    bias2p, g2p, b2p = bias2[p2], bn2_g[p2], bn2_b[p2]
    rm2p, rv2p = bn2_rm[p2], bn2_rv[p2]
    fc1_wp = fc1_w[:HC2][p2]
    fc1_wg = jnp.pad(fc1_w[HC2:], ((0, 12), (0, 0)))
    globp = jnp.pad(global_feat, ((0, 0), (0, 12)))
    fc2_wp = jnp.pad(fc2_w, ((0, 0), (0, 7)))
    fc2_bp = jnp.pad(fc2_b, (0, 7))

    src_pad = jnp.pad(edge_index[0], (0, NW * EPW - E))
    dst_pad = jnp.pad(edge_index[1], (0, NW * EPW - E))

    hist = _bin_hist(dst_pad, E, NB, EPW)
    esrc, edst, pstart, cnt = _bin_place(src_pad, dst_pad, hist, E, NB,
                                         EPW, EPAD)

    xl1, xr1 = _mm_pair(x, Wl1p, Wr1p)
    num1, den1 = _gat_edge(xl1, xr1, att1p, esrc, edst, pstart, cnt,
                           H1, C1, B, NB2, N)
    h1 = _epilogue(num1.reshape(NPAD, HC1), den1.reshape(NPAD, 16),
                   bias1p, g1p, b1p, rm1p, rv1p)

    xl2, xr2 = _mm_pair(h1, Wl2p, Wr2p)
    num2, den2 = _gat_edge(xl2, xr2, att2p, esrc, edst, pstart, cnt,
                           H2, C2, B, NB2, N)
    h2 = _epilogue(num2.reshape(NPAD, HC2), den2.reshape(NPAD, 16),
                   bias2p, g2p, b2p, rm2p, rv2p)

    NPOOL = _roundup(N, 256)
    hpool = jnp.pad(h2[:N], ((0, NPOOL - N), (0, 0)))
    bpool = jnp.pad(batch, (0, NPOOL - N), constant_values=G)
    batch3 = bpool.reshape(NPOOL // 256, 1, 256)
    sums, cnts = _pool(hpool, batch3, G)
    out = _head(sums, cnts, globp, fc1_wp, fc1_wg, fc1_b, fc2_wp, fc2_bp)
    return out[:, 0]


# speedup vs baseline: 14.1611x; 1.2821x over previous
import functools

import numpy as np
import jax
import jax.numpy as jnp
from jax import lax
from jax.experimental import pallas as pl
from jax.experimental.pallas import tpu as pltpu
from jax.experimental.pallas import tpu_sc as plsc

NW = 32
LBW = 6
W_BKT = 1 << LBW
BB = 2048
EBLK = 16
EWIN = 2048
RANK_BASE = 1


def _roundup(x, m):
    return (x + m - 1) // m * m


def _sc_mesh():
    return plsc.VectorSubcoreMesh(core_axis_name="c", subcore_axis_name="s")


_SC_PARAMS = pltpu.CompilerParams(needs_layout_passes=False)


def _perm_idx(H, C):
    HC = H * C
    k = np.arange(HC)
    j, l = k // 16, k % 16
    h = l % H
    c = j * (16 // H) + l // H
    return (h * C + c).astype(np.int32)


def _bin_hist(dst_pad, E, NB, EPW):
    nblk = EPW // BB

    @functools.partial(
        pl.kernel, mesh=_sc_mesh(), compiler_params=_SC_PARAMS,
        out_type=jax.ShapeDtypeStruct((NW, NB), jnp.int32),
        scratch_types=[
            pltpu.VMEM((BB,), jnp.int32),
            pltpu.VMEM((NB,), jnp.int32),
        ],
    )
    def k(dst_hbm, hist_hbm, buf_v, hist_v):
        wid = lax.axis_index("s") * 2 + lax.axis_index("c")
        wstart = wid * EPW
        zero16 = jnp.zeros((16,), jnp.int32)

        def zbody(i, c):
            hist_v[pl.ds(i * 16, 16)] = zero16
            return c
        lax.fori_loop(0, NB // 16, zbody, 0, unroll=True)

        iota = lax.iota(jnp.int32, 16)

        def blk(kb, c):
            pltpu.sync_copy(dst_hbm.at[pl.ds(wstart + kb * BB, BB)], buf_v)

            def chunk(i, c2):
                d = buf_v[pl.ds(i * 16, 16)]
                gidx = wstart + kb * BB + i * 16 + iota
                b = jnp.where(gidx < E, d >> LBW, NB - 1)
                rank, lastm = plsc.scan_count(b)
                plsc.addupdate_scatter(
                    hist_v, [b], rank + (1 - RANK_BASE), mask=lastm)
                return c2
            return lax.fori_loop(0, BB // 16, chunk, c)
        lax.fori_loop(0, nblk, blk, 0)
        pltpu.sync_copy(hist_v, hist_hbm.at[wid])
    return k(dst_pad)


def _bin_place(src_pad, dst_pad, hist, E, NB, EPW, EPAD):
    nblk = EPW // BB
    NB2 = NB + 16

    @functools.partial(
        pl.kernel, mesh=_sc_mesh(), compiler_params=_SC_PARAMS,
        out_type=(
            jax.ShapeDtypeStruct((EPAD,), jnp.int32),
            jax.ShapeDtypeStruct((EPAD,), jnp.int32),
            jax.ShapeDtypeStruct((NB2,), jnp.int32),
            jax.ShapeDtypeStruct((NB2,), jnp.int32),
        ),
        scratch_types=[
            pltpu.VMEM((NW, NB), jnp.int32),
            pltpu.VMEM((NB2,), jnp.int32),
            pltpu.VMEM((NB2,), jnp.int32),
            pltpu.VMEM((NB2,), jnp.int32),
            pltpu.VMEM((BB,), jnp.int32),
            pltpu.VMEM((BB,), jnp.int32),
            pltpu.VMEM((BB,), jnp.int32),
            pltpu.SemaphoreType.DMA,
            pltpu.SemaphoreType.DMA,
        ],
    )
    def k(src_hbm, dst_hbm, hist_hbm, esrc_hbm, edst_hbm, pstart_hbm,
          cnt_hbm, histl_v, off_v, pst_v, tot_v, sbuf_v, dbuf_v, pos_v,
          sem1, sem2):
        wid = lax.axis_index("s") * 2 + lax.axis_index("c")
        wstart = wid * EPW
        pltpu.sync_copy(hist_hbm, histl_v)
        zero16 = jnp.zeros((16,), jnp.int32)
        for cz in range(NB2 // 16):
            off_v[pl.ds(cz * 16, 16)] = zero16
            pst_v[pl.ds(cz * 16, 16)] = zero16
            tot_v[pl.ds(cz * 16, 16)] = zero16

        carry = jnp.int32(0)
        for c in range(NB // 16):
            sl = pl.ds(c * 16, 16)

            def addrow(wp, acc, _sl=sl):
                return acc + histl_v[wp, _sl]
            tot = lax.fori_loop(0, NW, addrow, zero16)
            mine = lax.fori_loop(0, wid, addrow, zero16)
            padded = jnp.bitwise_and(tot + 15, jnp.int32(-16))
            cums = plsc.cumsum(padded)
            pstart_c = cums - padded + carry
            off_v[sl] = pstart_c + mine
            pst_v[sl] = pstart_c
            tot_v[sl] = tot
            carry = carry + cums[15]

        @pl.when(wid == 0)
        def _():
            pltpu.sync_copy(pst_v, pstart_hbm)
            pltpu.sync_copy(tot_v, cnt_hbm)

        iota = lax.iota(jnp.int32, 16)

        def blk(kb, c):
            base = wstart + kb * BB
            pltpu.sync_copy(src_hbm.at[pl.ds(base, BB)], sbuf_v)
            pltpu.sync_copy(dst_hbm.at[pl.ds(base, BB)], dbuf_v)

            def chunk(i, c2):
                d = dbuf_v[pl.ds(i * 16, 16)]
                gidx = base + i * 16 + iota
                b = jnp.where(gidx < E, d >> LBW, NB - 1)
                bs = plsc.load_gather(off_v, [b])
                rank, lastm = plsc.scan_count(b)
                pos = jnp.minimum(bs + (rank - RANK_BASE), EPAD - 1)
                plsc.store_scatter(off_v, [b], pos + 1, mask=lastm)
                pos_v[pl.ds(i * 16, 16)] = pos
                return c2
            lax.fori_loop(0, BB // 16, chunk, c)
            cp1 = pltpu.async_copy(sbuf_v, esrc_hbm.at[pos_v], sem1)
            cp2 = pltpu.async_copy(dbuf_v, edst_hbm.at[pos_v], sem2)
            cp1.wait()
            cp2.wait()
            return c
        lax.fori_loop(0, nblk, blk, 0)
    return k(src_pad, dst_pad, hist)


def _gat_edge(xl, xr_flat, att_p, esrc, edst, pstart, cnt, H, C, B, NB2,
              NLIM):
    HC = H * C
    NPAD = B * W_BKT
    NCH = HC // 16
    nbk_max = _roundup(B, NW) // NW

    @functools.partial(
        pl.kernel, mesh=_sc_mesh(), compiler_params=_SC_PARAMS,
        out_type=(
            jax.ShapeDtypeStruct((NPAD * HC,), jnp.float32),
            jax.ShapeDtypeStruct((NPAD * 16,), jnp.float32),
        ),
        scratch_types=[
            pltpu.VMEM((W_BKT * HC,), jnp.float32),
            pltpu.VMEM((W_BKT * HC,), jnp.float32),
            pltpu.VMEM((W_BKT * 16,), jnp.float32),
            pltpu.VMEM((EBLK, HC), jnp.float32),
            pltpu.VMEM((EBLK, HC), jnp.float32),
            pltpu.VMEM((EBLK,), jnp.int32),
            pltpu.VMEM((EBLK,), jnp.int32),
            pltpu.VMEM((HC,), jnp.float32),
            pltpu.VMEM((EWIN + 16,), jnp.int32),
            pltpu.VMEM((EWIN + 16,), jnp.int32),
            pltpu.VMEM((32,), jnp.float32),
            pltpu.VMEM((NB2,), jnp.int32),
            pltpu.VMEM((NB2,), jnp.int32),
            pltpu.SemaphoreType.DMA,
            pltpu.SemaphoreType.DMA,
            pltpu.SemaphoreType.DMA,
        ],
    )
    def k(xl_hbm, xr_hbm, att_hbm, esrc_hbm, edst_hbm, pstart_hbm, cnt_hbm,
          num_hbm, den_hbm, acc_v, xrf_v, den_v, xlb0_v, xlb1_v, gi0_v,
          gi1_v, att_v, swin_v, dwin_v, rot_v, pst_v, cnt_v, semx, sem0,
          sem1):
        wid = lax.axis_index("s") * 2 + lax.axis_index("c")
        pltpu.sync_copy(att_hbm, att_v)
        pltpu.sync_copy(pstart_hbm, pst_v)
        pltpu.sync_copy(cnt_hbm, cnt_v)
        zf16 = jnp.zeros((16,), jnp.float32)
        xlb = (xlb0_v, xlb1_v)
        gi = (gi0_v, gi1_v)
        sems = (sem0, sem1)

        def bucket_body(t, c):
            b = wid + t * NW

            @pl.when(b < B)
            def _():
                bstart = pl.multiple_of(pst_v[pl.ds(b, 16)][0], 16)
                bcnt = cnt_v[pl.ds(b, 16)][0]
                cpx = pltpu.async_copy(
                    xr_hbm.at[pl.ds(b * W_BKT * HC, W_BKT * HC)], xrf_v,
                    semx)

                def z1(i, c2):
                    acc_v[pl.ds(i * 16, 16)] = zf16
                    return c2
                lax.fori_loop(0, W_BKT * NCH, z1, 0)

                def z2(i, c2):
                    den_v[pl.ds(i * 16, 16)] = zf16
                    return c2
                lax.fori_loop(0, W_BKT, z2, 0)
                cpx.wait()

                def prefetch(q, ph):
                    for r in range(EBLK // 16):
                        sq = pl.ds(q * EBLK + r * 16, 16)
                        gi[ph][pl.ds(r * 16, 16)] = jnp.clip(
                            swin_v[sq], 0, NLIM - 1)
                    pltpu.async_copy(xl_hbm.at[gi[ph]], xlb[ph], sems[ph])

                def window(wi, c2):
                    wbase = bstart + wi * EWIN
                    pltpu.sync_copy(esrc_hbm.at[pl.ds(wbase, EWIN)],
                                    swin_v.at[pl.ds(0, EWIN)])
                    pltpu.sync_copy(edst_hbm.at[pl.ds(wbase, EWIN)],
                                    dwin_v.at[pl.ds(0, EWIN)])
                    nloc = jnp.minimum(bcnt - wi * EWIN, EWIN)
                    nblk = pl.cdiv(nloc, EBLK)
                    prefetch(0, 0)

                    def compute(q, ph):
                        pltpu.make_async_copy(xl_hbm.at[gi[ph]], xlb[ph],
                                              sems[ph]).wait()
                        nleft = jnp.minimum(nloc - q * EBLK, EBLK)
                        xlq = xlb[ph]

                        def edge(e, c4):
                            dstoff = (dwin_v[pl.ds(q * EBLK + e, 16)][0]
                                      - b * W_BKT)
                            rbase = dstoff * HC
                            a0 = zf16
                            a1 = zf16
                            a2 = zf16
                            a3 = zf16
                            for j in range(NCH):
                                sl = pl.ds(j * 16, 16)
                                s = xlq[e, sl] + xrf_v[pl.ds(rbase + j * 16,
                                                             16)]
                                tlr = jnp.maximum(s, 0.2 * s)
                                prod = tlr * att_v[sl]
                                if j % 4 == 0:
                                    a0 = a0 + prod
                                elif j % 4 == 1:
                                    a1 = a1 + prod
                                elif j % 4 == 2:
                                    a2 = a2 + prod
                                else:
                                    a3 = a3 + prod
                            acc = (a0 + a1) + (a2 + a3)
                            step = H
                            while step < 16:
                                rot_v[pl.ds(0, 16)] = acc
                                rot_v[pl.ds(16, 16)] = acc
                                acc = acc + rot_v[pl.ds(step, 16)]
                                step *= 2
                            p = jnp.exp(acc)
                            den_v[pl.ds(dstoff * 16, 16)] += p
                            for j in range(NCH):
                                acc_v[pl.ds(rbase + j * 16, 16)] += (
                                    p * xlq[e, pl.ds(j * 16, 16)])
                            return c4
                        lax.fori_loop(0, nleft, edge, 0)

                    def blkpair(qq, c3):
                        for ph in range(2):
                            q = qq * 2 + ph

                            @pl.when(q < nblk)
                            def _():
                                @pl.when(q + 1 < nblk)
                                def _():
                                    prefetch(q + 1, 1 - ph)
                                compute(q, ph)
                        return c3
                    lax.fori_loop(0, pl.cdiv(nblk, 2), blkpair, 0)
                    return c2
                lax.fori_loop(0, pl.cdiv(bcnt, EWIN), window, 0)
                pltpu.sync_copy(acc_v,
                                num_hbm.at[pl.ds(b * W_BKT * HC, W_BKT * HC)])
                pltpu.sync_copy(den_v,
                                den_hbm.at[pl.ds(b * W_BKT * 16, W_BKT * 16)])
            return c
        lax.fori_loop(0, nbk_max, bucket_body, 0)
    return k(xl, xr_flat, att_p, esrc, edst, pstart, cnt)


def _mm_pair_body(x_ref, wl_ref, wr_ref, ol_ref, or_ref):
    xb = x_ref[...]
    ol_ref[...] = jnp.dot(xb, wl_ref[...], preferred_element_type=jnp.float32)
    or_ref[...] = jnp.dot(xb, wr_ref[...], preferred_element_type=jnp.float32)


def _mm_pair(x, wl, wr):
    M, K = x.shape
    HC = wl.shape[1]
    BLK = 1024
    grid = (pl.cdiv(M, BLK),)
    return pl.pallas_call(
        _mm_pair_body,
        grid=grid,
        in_specs=[
            pl.BlockSpec((BLK, K), lambda i: (i, 0)),
            pl.BlockSpec((K, HC), lambda i: (0, 0)),
            pl.BlockSpec((K, HC), lambda i: (0, 0)),
        ],
        out_specs=[
            pl.BlockSpec((BLK, HC), lambda i: (i, 0)),
            pl.BlockSpec((BLK, HC), lambda i: (i, 0)),
        ],
        out_shape=[
            jax.ShapeDtypeStruct((M, HC), jnp.float32),
            jax.ShapeDtypeStruct((M, HC), jnp.float32),
        ],
    )(x, wl, wr)


def _epilogue_body(num_ref, den_ref, bias_ref, scale_ref, shift_ref, o_ref,
                   *, reps):
    num = num_ref[...]
    den = jnp.concatenate([den_ref[...]] * reps, axis=1)
    h = num / (den + 1e-16) + bias_ref[...]
    o_ref[...] = jnp.maximum(h * scale_ref[...] + shift_ref[...], 0.0)


def _epilogue(num, den, bias, bn_g, bn_b, bn_rm, bn_rv):
    n, HC = num.shape
    inv = bn_g / jnp.sqrt(bn_rv + 1e-5)
    shift = bn_b - bn_rm * inv
    BLK = 1024
    return pl.pallas_call(
        functools.partial(_epilogue_body, reps=HC // 16),
        grid=(pl.cdiv(n, BLK),),
        in_specs=[
            pl.BlockSpec((BLK, HC), lambda i: (i, 0)),
            pl.BlockSpec((BLK, 16), lambda i: (i, 0)),
            pl.BlockSpec((1, HC), lambda i: (0, 0)),
            pl.BlockSpec((1, HC), lambda i: (0, 0)),
            pl.BlockSpec((1, HC), lambda i: (0, 0)),
        ],
        out_specs=pl.BlockSpec((BLK, HC), lambda i: (i, 0)),
        out_shape=jax.ShapeDtypeStruct((n, HC), jnp.float32),
    )(num, den, bias.reshape(1, HC), inv.reshape(1, HC), shift.reshape(1, HC))


def _pool_body(b_ref, h_ref, sum_ref, cnt_ref, *, G):
    i = pl.program_id(0)
    bb = b_ref[0, 0, :]
    onehot = (lax.broadcasted_iota(jnp.int32, (G, bb.shape[0]), 0)
              == bb[None, :]).astype(jnp.float32)

    @pl.when(i == 0)
    def _():
        sum_ref[...] = jnp.zeros_like(sum_ref)
        cnt_ref[...] = jnp.zeros_like(cnt_ref)

    sum_ref[...] += jnp.dot(onehot, h_ref[...],
                            preferred_element_type=jnp.float32)
    cnt_ref[...] += jnp.dot(onehot,
                            jnp.ones((bb.shape[0], 128), jnp.float32),
                            preferred_element_type=jnp.float32)


def _pool(h, batch3, G):
    NP, HC = h.shape
    BLK = 256
    nblk = NP // BLK
    return pl.pallas_call(
        functools.partial(_pool_body, G=G),
        grid=(nblk,),
        in_specs=[
            pl.BlockSpec((1, 1, BLK), lambda i: (i, 0, 0)),
            pl.BlockSpec((BLK, HC), lambda i: (i, 0)),
        ],
        out_specs=[
            pl.BlockSpec((G, HC), lambda i: (0, 0)),
            pl.BlockSpec((G, 128), lambda i: (0, 0)),
        ],
        out_shape=[
            jax.ShapeDtypeStruct((G, HC), jnp.float32),
            jax.ShapeDtypeStruct((G, 128), jnp.float32),
        ],
    )(batch3, h)


def _head_body(sum_ref, cnt_ref, glob_ref, w1a_ref, w1b_ref, b1_ref,
               w2_ref, b2_ref, o_ref):
    cnt = jnp.maximum(cnt_ref[:, 0:1], 1.0)
    pooled = sum_ref[...] / cnt
    z = (jnp.dot(pooled, w1a_ref[...], preferred_element_type=jnp.float32)
         + jnp.dot(glob_ref[...], w1b_ref[...],
                   preferred_element_type=jnp.float32)
         + b1_ref[...])
    z = jnp.maximum(z, 0.0)
    o_ref[...] = jnp.dot(z, w2_ref[...],
                         preferred_element_type=jnp.float32) + b2_ref[...]


def _head(sums, cnts, globp, w1a, w1b, b1, w2, b2):
    G = sums.shape[0]
    return pl.pallas_call(
        _head_body,
        out_shape=jax.ShapeDtypeStruct((G, 8), jnp.float32),
    )(sums, cnts, globp, w1a, w1b, b1.reshape(1, -1), w2, b2.reshape(1, -1))


def kernel(x, edge_index, batch, global_feat, Wl1, Wr1, att1, bias1, bn1_g,
           bn1_b, bn1_rm, bn1_rv, Wl2, Wr2, att2, bias2, bn2_g, bn2_b,
           bn2_rm, bn2_rv, fc1_w, fc1_b, fc2_w, fc2_b):
    N = x.shape[0]
    E = edge_index.shape[1]
    G = global_feat.shape[0]
    H1, C1 = att1.shape
    H2, C2 = att2.shape
    HC1, HC2 = H1 * C1, H2 * C2

    B = -(-N // W_BKT)
    NB = _roundup(B + 1, 16)
    NB2 = NB + 16
    NPAD = B * W_BKT
    EPW = _roundup(-(-E // NW), BB)
    EPAD = NW * EPW + 16 * NB + EWIN

    p1 = jnp.asarray(_perm_idx(H1, C1))
    p2 = jnp.asarray(_perm_idx(H2, C2))

    Wl1p, Wr1p = Wl1[:, p1], Wr1[:, p1]
    att1p = att1.reshape(HC1)[p1]
    bias1p, g1p, b1p = bias1[p1], bn1_g[p1], bn1_b[p1]
    rm1p, rv1p = bn1_rm[p1], bn1_rv[p1]
    Wl2p, Wr2p = Wl2[p1][:, p2], Wr2[p1][:, p2]
    att2p = att2.reshape(HC2)[p2]
    bias2p, g2p, b2p = bias2[p2], bn2_g[p2], bn2_b[p2]
    rm2p, rv2p = bn2_rm[p2], bn2_rv[p2]
    fc1_wp = fc1_w[:HC2][p2]
    fc1_wg = jnp.pad(fc1_w[HC2:], ((0, 12), (0, 0)))
    globp = jnp.pad(global_feat, ((0, 0), (0, 12)))
    fc2_wp = jnp.pad(fc2_w, ((0, 0), (0, 7)))
    fc2_bp = jnp.pad(fc2_b, (0, 7))

    src_pad = jnp.pad(edge_index[0], (0, NW * EPW - E))
    dst_pad = jnp.pad(edge_index[1], (0, NW * EPW - E))

    hist = _bin_hist(dst_pad, E, NB, EPW)
    esrc, edst, pstart, cnt = _bin_place(src_pad, dst_pad, hist, E, NB,
                                         EPW, EPAD)

    xpad = jnp.pad(x, ((0, NPAD - N), (0, 0)))
    xl1, xr1 = _mm_pair(xpad, Wl1p, Wr1p)
    num1, den1 = _gat_edge(xl1, xr1.reshape(-1), att1p, esrc, edst, pstart,
                           cnt, H1, C1, B, NB2, N)
    h1 = _epilogue(num1.reshape(NPAD, HC1), den1.reshape(NPAD, 16),
                   bias1p, g1p, b1p, rm1p, rv1p)

    xl2, xr2 = _mm_pair(h1, Wl2p, Wr2p)
    num2, den2 = _gat_edge(xl2, xr2.reshape(-1), att2p, esrc, edst, pstart,
                           cnt, H2, C2, B, NB2, N)
    h2 = _epilogue(num2.reshape(NPAD, HC2), den2.reshape(NPAD, 16),
                   bias2p, g2p, b2p, rm2p, rv2p)

    NPOOL = _roundup(N, 256)
    hpool = jnp.pad(h2[:N], ((0, NPOOL - N), (0, 0)))
    bpool = jnp.pad(batch, (0, NPOOL - N), constant_values=G)
    batch3 = bpool.reshape(NPOOL // 256, 1, 256)
    sums, cnts = _pool(hpool, batch3, G)
    out = _head(sums, cnts, globp, fc1_wp, fc1_wg, fc1_b, fc2_wp, fc2_bp)
    return out[:, 0]


# speedup vs baseline: 15.0937x; 1.0659x over previous
import functools

import numpy as np
import jax
import jax.numpy as jnp
from jax import lax
from jax.experimental import pallas as pl
from jax.experimental.pallas import tpu as pltpu
from jax.experimental.pallas import tpu_sc as plsc

NW = 32
LBW = 6
W_BKT = 1 << LBW
BB = 2048
EBLK = 16
EWIN = 2048
RANK_BASE = 1


def _roundup(x, m):
    return (x + m - 1) // m * m


def _sc_mesh():
    return plsc.VectorSubcoreMesh(core_axis_name="c", subcore_axis_name="s")


_SC_PARAMS = pltpu.CompilerParams(needs_layout_passes=False)


def _perm_idx(H, C):
    HC = H * C
    k = np.arange(HC)
    j, l = k // 16, k % 16
    h = l % H
    c = j * (16 // H) + l // H
    return (h * C + c).astype(np.int32)


def _bin_hist(dst_pad, E, NB, EPW):
    nblk = EPW // BB

    @functools.partial(
        pl.kernel, mesh=_sc_mesh(), compiler_params=_SC_PARAMS,
        out_type=jax.ShapeDtypeStruct((NW, NB), jnp.int32),
        scratch_types=[
            pltpu.VMEM((BB,), jnp.int32),
            pltpu.VMEM((NB,), jnp.int32),
        ],
    )
    def k(dst_hbm, hist_hbm, buf_v, hist_v):
        wid = lax.axis_index("s") * 2 + lax.axis_index("c")
        wstart = wid * EPW
        zero16 = jnp.zeros((16,), jnp.int32)

        def zbody(i, c):
            hist_v[pl.ds(i * 16, 16)] = zero16
            return c
        lax.fori_loop(0, NB // 16, zbody, 0, unroll=True)

        iota = lax.iota(jnp.int32, 16)

        def blk(kb, c):
            pltpu.sync_copy(dst_hbm.at[pl.ds(wstart + kb * BB, BB)], buf_v)

            def chunk(i, c2):
                d = buf_v[pl.ds(i * 16, 16)]
                gidx = wstart + kb * BB + i * 16 + iota
                b = jnp.where(gidx < E, d >> LBW, NB - 1)
                rank, lastm = plsc.scan_count(b)
                plsc.addupdate_scatter(
                    hist_v, [b], rank + (1 - RANK_BASE), mask=lastm)
                return c2
            return lax.fori_loop(0, BB // 16, chunk, c)
        lax.fori_loop(0, nblk, blk, 0)
        pltpu.sync_copy(hist_v, hist_hbm.at[wid])
    return k(dst_pad)


def _bin_place(src_pad, dst_pad, hist, E, NB, EPW, EPAD):
    nblk = EPW // BB
    NB2 = NB + 16

    @functools.partial(
        pl.kernel, mesh=_sc_mesh(), compiler_params=_SC_PARAMS,
        out_type=(
            jax.ShapeDtypeStruct((EPAD,), jnp.int32),
            jax.ShapeDtypeStruct((EPAD,), jnp.int32),
            jax.ShapeDtypeStruct((NB2,), jnp.int32),
            jax.ShapeDtypeStruct((NB2,), jnp.int32),
        ),
        scratch_types=[
            pltpu.VMEM((NW, NB), jnp.int32),
            pltpu.VMEM((NB2,), jnp.int32),
            pltpu.VMEM((NB2,), jnp.int32),
            pltpu.VMEM((NB2,), jnp.int32),
            pltpu.VMEM((BB,), jnp.int32),
            pltpu.VMEM((BB,), jnp.int32),
            pltpu.VMEM((BB,), jnp.int32),
            pltpu.SemaphoreType.DMA,
            pltpu.SemaphoreType.DMA,
        ],
    )
    def k(src_hbm, dst_hbm, hist_hbm, esrc_hbm, edst_hbm, pstart_hbm,
          cnt_hbm, histl_v, off_v, pst_v, tot_v, sbuf_v, dbuf_v, pos_v,
          sem1, sem2):
        wid = lax.axis_index("s") * 2 + lax.axis_index("c")
        wstart = wid * EPW
        pltpu.sync_copy(hist_hbm, histl_v)
        zero16 = jnp.zeros((16,), jnp.int32)
        for cz in range(NB2 // 16):
            off_v[pl.ds(cz * 16, 16)] = zero16
            pst_v[pl.ds(cz * 16, 16)] = zero16
            tot_v[pl.ds(cz * 16, 16)] = zero16

        carry = jnp.int32(0)
        for c in range(NB // 16):
            sl = pl.ds(c * 16, 16)

            def addrow(wp, acc, _sl=sl):
                return acc + histl_v[wp, _sl]
            tot = lax.fori_loop(0, NW, addrow, zero16)
            mine = lax.fori_loop(0, wid, addrow, zero16)
            padded = jnp.bitwise_and(tot + 15, jnp.int32(-16))
            cums = plsc.cumsum(padded)
            pstart_c = cums - padded + carry
            off_v[sl] = pstart_c + mine
            pst_v[sl] = pstart_c
            tot_v[sl] = tot
            carry = carry + cums[15]

        @pl.when(wid == 0)
        def _():
            pltpu.sync_copy(pst_v, pstart_hbm)
            pltpu.sync_copy(tot_v, cnt_hbm)

        iota = lax.iota(jnp.int32, 16)

        def blk(kb, c):
            base = wstart + kb * BB
            pltpu.sync_copy(src_hbm.at[pl.ds(base, BB)], sbuf_v)
            pltpu.sync_copy(dst_hbm.at[pl.ds(base, BB)], dbuf_v)

            def chunk(i, c2):
                d = dbuf_v[pl.ds(i * 16, 16)]
                gidx = base + i * 16 + iota
                b = jnp.where(gidx < E, d >> LBW, NB - 1)
                bs = plsc.load_gather(off_v, [b])
                rank, lastm = plsc.scan_count(b)
                pos = jnp.minimum(bs + (rank - RANK_BASE), EPAD - 1)
                plsc.store_scatter(off_v, [b], pos + 1, mask=lastm)
                pos_v[pl.ds(i * 16, 16)] = pos
                return c2
            lax.fori_loop(0, BB // 16, chunk, c)
            cp1 = pltpu.async_copy(sbuf_v, esrc_hbm.at[pos_v], sem1)
            cp2 = pltpu.async_copy(dbuf_v, edst_hbm.at[pos_v], sem2)
            cp1.wait()
            cp2.wait()
            return c
        lax.fori_loop(0, nblk, blk, 0)
    return k(src_pad, dst_pad, hist)


def _gat_edge(xl, xr_flat, att_p, esrc, edst, pstart, cnt, H, C, B, NB2,
              NLIM):
    HC = H * C
    NPAD = B * W_BKT
    NCH = HC // 16
    nbk_max = _roundup(B, NW) // NW

    @functools.partial(
        pl.kernel, mesh=_sc_mesh(), compiler_params=_SC_PARAMS,
        out_type=(
            jax.ShapeDtypeStruct((NPAD * HC,), jnp.float32),
            jax.ShapeDtypeStruct((NPAD * 16,), jnp.float32),
        ),
        scratch_types=[
            pltpu.VMEM((W_BKT * HC,), jnp.float32),
            pltpu.VMEM((W_BKT * HC,), jnp.float32),
            pltpu.VMEM((W_BKT * 16,), jnp.float32),
            pltpu.VMEM((EBLK, HC), jnp.float32),
            pltpu.VMEM((EBLK, HC), jnp.float32),
            pltpu.VMEM((EBLK,), jnp.int32),
            pltpu.VMEM((EBLK,), jnp.int32),
            pltpu.VMEM((HC,), jnp.float32),
            pltpu.VMEM((EWIN + 16,), jnp.int32),
            pltpu.VMEM((EWIN + 16,), jnp.int32),
            pltpu.VMEM((16,), jnp.float32),
            pltpu.VMEM((16,), jnp.float32),
            pltpu.VMEM((NB2,), jnp.int32),
            pltpu.VMEM((NB2,), jnp.int32),
            pltpu.SemaphoreType.DMA,
            pltpu.SemaphoreType.DMA,
            pltpu.SemaphoreType.DMA,
        ],
    )
    def k(xl_hbm, xr_hbm, att_hbm, esrc_hbm, edst_hbm, pstart_hbm, cnt_hbm,
          num_hbm, den_hbm, acc_v, xrf_v, den_v, xlb0_v, xlb1_v, gi0_v,
          gi1_v, att_v, swin_v, dwin_v, rot0_v, rot1_v, pst_v, cnt_v, semx,
          sem0, sem1):
        wid = lax.axis_index("s") * 2 + lax.axis_index("c")
        pltpu.sync_copy(att_hbm, att_v)
        pltpu.sync_copy(pstart_hbm, pst_v)
        pltpu.sync_copy(cnt_hbm, cnt_v)
        zf16 = jnp.zeros((16,), jnp.float32)
        xlb = (xlb0_v, xlb1_v)
        gi = (gi0_v, gi1_v)
        sems = (sem0, sem1)
        iota = lax.iota(jnp.int32, 16)
        rotidx = []
        step = H
        while step < 16:
            rotidx.append(jnp.bitwise_and(iota + step, 15))
            step *= 2
        attc = [att_v[pl.ds(j * 16, 16)] for j in range(NCH)]

        def bucket_body(t, c):
            b = wid + t * NW

            @pl.when(b < B)
            def _():
                bstart = pl.multiple_of(pst_v[pl.ds(b, 16)][0], 16)
                bcnt = cnt_v[pl.ds(b, 16)][0]
                cpx = pltpu.async_copy(
                    xr_hbm.at[pl.ds(b * W_BKT * HC, W_BKT * HC)], xrf_v,
                    semx)

                def z1(i, c2):
                    acc_v[pl.ds(i * 16, 16)] = zf16
                    return c2
                lax.fori_loop(0, W_BKT * NCH, z1, 0)

                def z2(i, c2):
                    den_v[pl.ds(i * 16, 16)] = zf16
                    return c2
                lax.fori_loop(0, W_BKT, z2, 0)
                cpx.wait()

                def prefetch(q, ph):
                    for r in range(EBLK // 16):
                        sq = pl.ds(q * EBLK + r * 16, 16)
                        gi[ph][pl.ds(r * 16, 16)] = jnp.clip(
                            swin_v[sq], 0, NLIM - 1)
                    pltpu.async_copy(xl_hbm.at[gi[ph]], xlb[ph], sems[ph])

                def window(wi, c2):
                    wbase = bstart + wi * EWIN
                    pltpu.sync_copy(esrc_hbm.at[pl.ds(wbase, EWIN)],
                                    swin_v.at[pl.ds(0, EWIN)])
                    pltpu.sync_copy(edst_hbm.at[pl.ds(wbase, EWIN)],
                                    dwin_v.at[pl.ds(0, EWIN)])
                    nloc = jnp.minimum(bcnt - wi * EWIN, EWIN)
                    nblk = pl.cdiv(nloc, EBLK)
                    prefetch(0, 0)

                    def compute(q, ph):
                        pltpu.make_async_copy(xl_hbm.at[gi[ph]], xlb[ph],
                                              sems[ph]).wait()
                        nleft = jnp.minimum(nloc - q * EBLK, EBLK)
                        xlq = xlb[ph]

                        def logits(e, dstoff):
                            rbase = dstoff * HC
                            a0 = zf16
                            a1 = zf16
                            a2 = zf16
                            a3 = zf16
                            for j in range(NCH):
                                sl = pl.ds(j * 16, 16)
                                s = xlq[e, sl] + xrf_v[pl.ds(rbase + j * 16,
                                                             16)]
                                tlr = jnp.maximum(s, 0.2 * s)
                                prod = tlr * attc[j]
                                if j % 4 == 0:
                                    a0 = a0 + prod
                                elif j % 4 == 1:
                                    a1 = a1 + prod
                                elif j % 4 == 2:
                                    a2 = a2 + prod
                                else:
                                    a3 = a3 + prod
                            return (a0 + a1) + (a2 + a3)

                        def fold_exp(acc, rot_v):
                            for ridx in rotidx:
                                rot_v[...] = acc
                                acc = acc + plsc.load_gather(rot_v, [ridx])
                            return jnp.exp(acc)

                        def accumulate(e, dstoff, p):
                            rbase = dstoff * HC
                            den_v[pl.ds(dstoff * 16, 16)] += p
                            for j in range(NCH):
                                acc_v[pl.ds(rbase + j * 16, 16)] += (
                                    p * xlq[e, pl.ds(j * 16, 16)])

                        def dst_of(e):
                            return (dwin_v[pl.ds(q * EBLK + e, 16)][0]
                                    - b * W_BKT)

                        def pair(ii, c4):
                            e0 = ii * 2
                            e1 = e0 + 1
                            d0 = dst_of(e0)
                            d1 = dst_of(e1)
                            p0 = fold_exp(logits(e0, d0), rot0_v)
                            p1 = fold_exp(logits(e1, d1), rot1_v)
                            accumulate(e0, d0, p0)
                            accumulate(e1, d1, p1)
                            return c4
                        lax.fori_loop(0, nleft // 2, pair, 0)

                        @pl.when(nleft % 2 == 1)
                        def _():
                            e = nleft - 1
                            d = dst_of(e)
                            accumulate(e, d, fold_exp(logits(e, d), rot0_v))

                    def blkpair(qq, c3):
                        for ph in range(2):
                            q = qq * 2 + ph

                            @pl.when(q < nblk)
                            def _():
                                @pl.when(q + 1 < nblk)
                                def _():
                                    prefetch(q + 1, 1 - ph)
                                compute(q, ph)
                        return c3
                    lax.fori_loop(0, pl.cdiv(nblk, 2), blkpair, 0)
                    return c2
                lax.fori_loop(0, pl.cdiv(bcnt, EWIN), window, 0)
                pltpu.sync_copy(acc_v,
                                num_hbm.at[pl.ds(b * W_BKT * HC, W_BKT * HC)])
                pltpu.sync_copy(den_v,
                                den_hbm.at[pl.ds(b * W_BKT * 16, W_BKT * 16)])
            return c
        lax.fori_loop(0, nbk_max, bucket_body, 0)
    return k(xl, xr_flat, att_p, esrc, edst, pstart, cnt)


def _mm_pair_body(x_ref, wl_ref, wr_ref, ol_ref, or_ref):
    xb = x_ref[...]
    ol_ref[...] = jnp.dot(xb, wl_ref[...], preferred_element_type=jnp.float32)
    or_ref[...] = jnp.dot(xb, wr_ref[...], preferred_element_type=jnp.float32)


def _mm_pair(x, wl, wr):
    M, K = x.shape
    HC = wl.shape[1]
    BLK = 1024
    grid = (pl.cdiv(M, BLK),)
    return pl.pallas_call(
        _mm_pair_body,
        grid=grid,
        in_specs=[
            pl.BlockSpec((BLK, K), lambda i: (i, 0)),
            pl.BlockSpec((K, HC), lambda i: (0, 0)),
            pl.BlockSpec((K, HC), lambda i: (0, 0)),
        ],
        out_specs=[
            pl.BlockSpec((BLK, HC), lambda i: (i, 0)),
            pl.BlockSpec((BLK, HC), lambda i: (i, 0)),
        ],
        out_shape=[
            jax.ShapeDtypeStruct((M, HC), jnp.float32),
            jax.ShapeDtypeStruct((M, HC), jnp.float32),
        ],
    )(x, wl, wr)


def _epilogue_body(num_ref, den_ref, bias_ref, scale_ref, shift_ref, o_ref,
                   *, reps):
    num = num_ref[...]
    den = jnp.concatenate([den_ref[...]] * reps, axis=1)
    h = num / (den + 1e-16) + bias_ref[...]
    o_ref[...] = jnp.maximum(h * scale_ref[...] + shift_ref[...], 0.0)


def _epilogue(num, den, bias, bn_g, bn_b, bn_rm, bn_rv):
    n, HC = num.shape
    inv = bn_g / jnp.sqrt(bn_rv + 1e-5)
    shift = bn_b - bn_rm * inv
    BLK = 1024
    return pl.pallas_call(
        functools.partial(_epilogue_body, reps=HC // 16),
        grid=(pl.cdiv(n, BLK),),
        in_specs=[
            pl.BlockSpec((BLK, HC), lambda i: (i, 0)),
            pl.BlockSpec((BLK, 16), lambda i: (i, 0)),
            pl.BlockSpec((1, HC), lambda i: (0, 0)),
            pl.BlockSpec((1, HC), lambda i: (0, 0)),
            pl.BlockSpec((1, HC), lambda i: (0, 0)),
        ],
        out_specs=pl.BlockSpec((BLK, HC), lambda i: (i, 0)),
        out_shape=jax.ShapeDtypeStruct((n, HC), jnp.float32),
    )(num, den, bias.reshape(1, HC), inv.reshape(1, HC), shift.reshape(1, HC))


def _pool_body(b_ref, h_ref, sum_ref, cnt_ref, *, G):
    i = pl.program_id(0)
    bb = b_ref[0, 0, :]
    onehot = (lax.broadcasted_iota(jnp.int32, (G, bb.shape[0]), 0)
              == bb[None, :]).astype(jnp.float32)

    @pl.when(i == 0)
    def _():
        sum_ref[...] = jnp.zeros_like(sum_ref)
        cnt_ref[...] = jnp.zeros_like(cnt_ref)

    sum_ref[...] += jnp.dot(onehot, h_ref[...],
                            preferred_element_type=jnp.float32)
    cnt_ref[...] += jnp.dot(onehot,
                            jnp.ones((bb.shape[0], 128), jnp.float32),
                            preferred_element_type=jnp.float32)


def _pool(h, batch3, G):
    NP, HC = h.shape
    BLK = 256
    nblk = NP // BLK
    return pl.pallas_call(
        functools.partial(_pool_body, G=G),
        grid=(nblk,),
        in_specs=[
            pl.BlockSpec((1, 1, BLK), lambda i: (i, 0, 0)),
            pl.BlockSpec((BLK, HC), lambda i: (i, 0)),
        ],
        out_specs=[
            pl.BlockSpec((G, HC), lambda i: (0, 0)),
            pl.BlockSpec((G, 128), lambda i: (0, 0)),
        ],
        out_shape=[
            jax.ShapeDtypeStruct((G, HC), jnp.float32),
            jax.ShapeDtypeStruct((G, 128), jnp.float32),
        ],
    )(batch3, h)


def _head_body(sum_ref, cnt_ref, glob_ref, w1a_ref, w1b_ref, b1_ref,
               w2_ref, b2_ref, o_ref):
    cnt = jnp.maximum(cnt_ref[:, 0:1], 1.0)
    pooled = sum_ref[...] / cnt
    z = (jnp.dot(pooled, w1a_ref[...], preferred_element_type=jnp.float32)
         + jnp.dot(glob_ref[...], w1b_ref[...],
                   preferred_element_type=jnp.float32)
         + b1_ref[...])
    z = jnp.maximum(z, 0.0)
    o_ref[...] = jnp.dot(z, w2_ref[...],
                         preferred_element_type=jnp.float32) + b2_ref[...]


def _head(sums, cnts, globp, w1a, w1b, b1, w2, b2):
    G = sums.shape[0]
    return pl.pallas_call(
        _head_body,
        out_shape=jax.ShapeDtypeStruct((G, 8), jnp.float32),
    )(sums, cnts, globp, w1a, w1b, b1.reshape(1, -1), w2, b2.reshape(1, -1))


def kernel(x, edge_index, batch, global_feat, Wl1, Wr1, att1, bias1, bn1_g,
           bn1_b, bn1_rm, bn1_rv, Wl2, Wr2, att2, bias2, bn2_g, bn2_b,
           bn2_rm, bn2_rv, fc1_w, fc1_b, fc2_w, fc2_b):
    N = x.shape[0]
    E = edge_index.shape[1]
    G = global_feat.shape[0]
    H1, C1 = att1.shape
    H2, C2 = att2.shape
    HC1, HC2 = H1 * C1, H2 * C2

    B = -(-N // W_BKT)
    NB = _roundup(B + 1, 16)
    NB2 = NB + 16
    NPAD = B * W_BKT
    EPW = _roundup(-(-E // NW), BB)
    EPAD = NW * EPW + 16 * NB + EWIN

    p1 = jnp.asarray(_perm_idx(H1, C1))
    p2 = jnp.asarray(_perm_idx(H2, C2))

    Wl1p, Wr1p = Wl1[:, p1], Wr1[:, p1]
    att1p = att1.reshape(HC1)[p1]
    bias1p, g1p, b1p = bias1[p1], bn1_g[p1], bn1_b[p1]
    rm1p, rv1p = bn1_rm[p1], bn1_rv[p1]
    Wl2p, Wr2p = Wl2[p1][:, p2], Wr2[p1][:, p2]
    att2p = att2.reshape(HC2)[p2]
    bias2p, g2p, b2p = bias2[p2], bn2_g[p2], bn2_b[p2]
    rm2p, rv2p = bn2_rm[p2], bn2_rv[p2]
    fc1_wp = fc1_w[:HC2][p2]
    fc1_wg = jnp.pad(fc1_w[HC2:], ((0, 12), (0, 0)))
    globp = jnp.pad(global_feat, ((0, 0), (0, 12)))
    fc2_wp = jnp.pad(fc2_w, ((0, 0), (0, 7)))
    fc2_bp = jnp.pad(fc2_b, (0, 7))

    src_pad = jnp.pad(edge_index[0], (0, NW * EPW - E))
    dst_pad = jnp.pad(edge_index[1], (0, NW * EPW - E))

    hist = _bin_hist(dst_pad, E, NB, EPW)
    esrc, edst, pstart, cnt = _bin_place(src_pad, dst_pad, hist, E, NB,
                                         EPW, EPAD)

    xpad = jnp.pad(x, ((0, NPAD - N), (0, 0)))
    xl1, xr1 = _mm_pair(xpad, Wl1p, Wr1p)
    num1, den1 = _gat_edge(xl1, xr1.reshape(-1), att1p, esrc, edst, pstart,
                           cnt, H1, C1, B, NB2, N)
    h1 = _epilogue(num1.reshape(NPAD, HC1), den1.reshape(NPAD, 16),
                   bias1p, g1p, b1p, rm1p, rv1p)

    xl2, xr2 = _mm_pair(h1, Wl2p, Wr2p)
    num2, den2 = _gat_edge(xl2, xr2.reshape(-1), att2p, esrc, edst, pstart,
                           cnt, H2, C2, B, NB2, N)
    h2 = _epilogue(num2.reshape(NPAD, HC2), den2.reshape(NPAD, 16),
                   bias2p, g2p, b2p, rm2p, rv2p)

    NPOOL = _roundup(N, 256)
    hpool = jnp.pad(h2[:N], ((0, NPOOL - N), (0, 0)))
    bpool = jnp.pad(batch, (0, NPOOL - N), constant_values=G)
    batch3 = bpool.reshape(NPOOL // 256, 1, 256)
    sums, cnts = _pool(hpool, batch3, G)
    out = _head(sums, cnts, globp, fc1_wp, fc1_wg, fc1_b, fc2_wp, fc2_bp)
    return out[:, 0]


# speedup vs baseline: 15.1681x; 1.0049x over previous
import functools

import numpy as np
import jax
import jax.numpy as jnp
from jax import lax
from jax.experimental import pallas as pl
from jax.experimental.pallas import tpu as pltpu
from jax.experimental.pallas import tpu_sc as plsc

NW = 32
LBW = 6
W_BKT = 1 << LBW
BB = 2048
EBLK = 32
EWIN = 2048
RANK_BASE = 1


def _roundup(x, m):
    return (x + m - 1) // m * m


def _sc_mesh():
    return plsc.VectorSubcoreMesh(core_axis_name="c", subcore_axis_name="s")


_SC_PARAMS = pltpu.CompilerParams(needs_layout_passes=False)


def _perm_idx(H, C):
    HC = H * C
    k = np.arange(HC)
    j, l = k // 16, k % 16
    h = l % H
    c = j * (16 // H) + l // H
    return (h * C + c).astype(np.int32)


def _bin_hist(dst_pad, E, NB, EPW):
    nblk = EPW // BB

    @functools.partial(
        pl.kernel, mesh=_sc_mesh(), compiler_params=_SC_PARAMS,
        out_type=jax.ShapeDtypeStruct((NW, NB), jnp.int32),
        scratch_types=[
            pltpu.VMEM((BB,), jnp.int32),
            pltpu.VMEM((NB,), jnp.int32),
        ],
    )
    def k(dst_hbm, hist_hbm, buf_v, hist_v):
        wid = lax.axis_index("s") * 2 + lax.axis_index("c")
        wstart = wid * EPW
        zero16 = jnp.zeros((16,), jnp.int32)

        def zbody(i, c):
            hist_v[pl.ds(i * 16, 16)] = zero16
            return c
        lax.fori_loop(0, NB // 16, zbody, 0, unroll=True)

        iota = lax.iota(jnp.int32, 16)

        def blk(kb, c):
            pltpu.sync_copy(dst_hbm.at[pl.ds(wstart + kb * BB, BB)], buf_v)

            def chunk(i, c2):
                d = buf_v[pl.ds(i * 16, 16)]
                gidx = wstart + kb * BB + i * 16 + iota
                b = jnp.where(gidx < E, d >> LBW, NB - 1)
                rank, lastm = plsc.scan_count(b)
                plsc.addupdate_scatter(
                    hist_v, [b], rank + (1 - RANK_BASE), mask=lastm)
                return c2
            return lax.fori_loop(0, BB // 16, chunk, c)
        lax.fori_loop(0, nblk, blk, 0)
        pltpu.sync_copy(hist_v, hist_hbm.at[wid])
    return k(dst_pad)


def _bin_place(src_pad, dst_pad, hist, E, NB, EPW, EPAD):
    nblk = EPW // BB
    NB2 = NB + 16

    @functools.partial(
        pl.kernel, mesh=_sc_mesh(), compiler_params=_SC_PARAMS,
        out_type=(
            jax.ShapeDtypeStruct((EPAD,), jnp.int32),
            jax.ShapeDtypeStruct((EPAD,), jnp.int32),
            jax.ShapeDtypeStruct((NB2,), jnp.int32),
            jax.ShapeDtypeStruct((NB2,), jnp.int32),
        ),
        scratch_types=[
            pltpu.VMEM((NW, NB), jnp.int32),
            pltpu.VMEM((NB2,), jnp.int32),
            pltpu.VMEM((NB2,), jnp.int32),
            pltpu.VMEM((NB2,), jnp.int32),
            pltpu.VMEM((BB,), jnp.int32),
            pltpu.VMEM((BB,), jnp.int32),
            pltpu.VMEM((BB,), jnp.int32),
            pltpu.VMEM((BB,), jnp.int32),
            pltpu.VMEM((BB,), jnp.int32),
            pltpu.VMEM((BB,), jnp.int32),
            pltpu.SemaphoreType.DMA,
            pltpu.SemaphoreType.DMA,
            pltpu.SemaphoreType.DMA,
            pltpu.SemaphoreType.DMA,
        ],
    )
    def k(src_hbm, dst_hbm, hist_hbm, esrc_hbm, edst_hbm, pstart_hbm,
          cnt_hbm, histl_v, off_v, pst_v, tot_v, sbuf0_v, dbuf0_v, pos0_v,
          sbuf1_v, dbuf1_v, pos1_v, sem1, sem2, sem3, sem4):
        wid = lax.axis_index("s") * 2 + lax.axis_index("c")
        wstart = wid * EPW
        pltpu.sync_copy(hist_hbm, histl_v)
        zero16 = jnp.zeros((16,), jnp.int32)
        for cz in range(NB2 // 16):
            off_v[pl.ds(cz * 16, 16)] = zero16
            pst_v[pl.ds(cz * 16, 16)] = zero16
            tot_v[pl.ds(cz * 16, 16)] = zero16

        carry = jnp.int32(0)
        for c in range(NB // 16):
            sl = pl.ds(c * 16, 16)

            def addrow(wp, acc, _sl=sl):
                return acc + histl_v[wp, _sl]
            tot = lax.fori_loop(0, NW, addrow, zero16)
            mine = lax.fori_loop(0, wid, addrow, zero16)
            padded = jnp.bitwise_and(tot + 15, jnp.int32(-16))
            cums = plsc.cumsum(padded)
            pstart_c = cums - padded + carry
            off_v[sl] = pstart_c + mine
            pst_v[sl] = pstart_c
            tot_v[sl] = tot
            carry = carry + cums[15]

        @pl.when(wid == 0)
        def _():
            pltpu.sync_copy(pst_v, pstart_hbm)
            pltpu.sync_copy(tot_v, cnt_hbm)

        iota = lax.iota(jnp.int32, 16)
        sbuf = (sbuf0_v, sbuf1_v)
        dbuf = (dbuf0_v, dbuf1_v)
        pos = (pos0_v, pos1_v)
        ssem = (sem1, sem3)
        dsem = (sem2, sem4)
        pend = [None, None]
        for kb in range(nblk):
            ph = kb % 2
            if pend[ph] is not None:
                pend[ph][0].wait()
                pend[ph][1].wait()
            base = wstart + kb * BB
            pltpu.sync_copy(src_hbm.at[pl.ds(base, BB)], sbuf[ph])
            pltpu.sync_copy(dst_hbm.at[pl.ds(base, BB)], dbuf[ph])

            def chunk(i, c2, _base=base, _ph=ph):
                d = dbuf[_ph][pl.ds(i * 16, 16)]
                gidx = _base + i * 16 + iota
                b = jnp.where(gidx < E, d >> LBW, NB - 1)
                bs = plsc.load_gather(off_v, [b])
                rank, lastm = plsc.scan_count(b)
                p = jnp.minimum(bs + (rank - RANK_BASE), EPAD - 1)
                plsc.store_scatter(off_v, [b], p + 1, mask=lastm)
                pos[_ph][pl.ds(i * 16, 16)] = p
                return c2
            lax.fori_loop(0, BB // 16, chunk, 0)
            cp1 = pltpu.async_copy(sbuf[ph], esrc_hbm.at[pos[ph]], ssem[ph])
            cp2 = pltpu.async_copy(dbuf[ph], edst_hbm.at[pos[ph]], dsem[ph])
            pend[ph] = (cp1, cp2)
        for ph in range(2):
            if pend[ph] is not None:
                pend[ph][0].wait()
                pend[ph][1].wait()
    return k(src_pad, dst_pad, hist)


def _gat_edge(xl, xr_flat, att_p, esrc, edst, pstart, cnt, H, C, B, NB2,
              NLIM):
    HC = H * C
    NPAD = B * W_BKT
    NCH = HC // 16
    nbk_max = _roundup(B, NW) // NW

    @functools.partial(
        pl.kernel, mesh=_sc_mesh(), compiler_params=_SC_PARAMS,
        out_type=(
            jax.ShapeDtypeStruct((NPAD * HC,), jnp.float32),
            jax.ShapeDtypeStruct((NPAD * 16,), jnp.float32),
        ),
        scratch_types=[
            pltpu.VMEM((W_BKT * HC,), jnp.float32),
            pltpu.VMEM((W_BKT * HC,), jnp.float32),
            pltpu.VMEM((W_BKT * 16,), jnp.float32),
            pltpu.VMEM((EBLK, HC), jnp.float32),
            pltpu.VMEM((EBLK, HC), jnp.float32),
            pltpu.VMEM((EBLK,), jnp.int32),
            pltpu.VMEM((EBLK,), jnp.int32),
            pltpu.VMEM((HC,), jnp.float32),
            pltpu.VMEM((EWIN + 16,), jnp.int32),
            pltpu.VMEM((EWIN + 16,), jnp.int32),
            pltpu.VMEM((16,), jnp.float32),
            pltpu.VMEM((16,), jnp.float32),
            pltpu.VMEM((NB2,), jnp.int32),
            pltpu.VMEM((NB2,), jnp.int32),
            pltpu.SemaphoreType.DMA,
            pltpu.SemaphoreType.DMA,
            pltpu.SemaphoreType.DMA,
        ],
    )
    def k(xl_hbm, xr_hbm, att_hbm, esrc_hbm, edst_hbm, pstart_hbm, cnt_hbm,
          num_hbm, den_hbm, acc_v, xrf_v, den_v, xlb0_v, xlb1_v, gi0_v,
          gi1_v, att_v, swin_v, dwin_v, rot0_v, rot1_v, pst_v, cnt_v, semx,
          sem0, sem1):
        wid = lax.axis_index("s") * 2 + lax.axis_index("c")
        pltpu.sync_copy(att_hbm, att_v)
        pltpu.sync_copy(pstart_hbm, pst_v)
        pltpu.sync_copy(cnt_hbm, cnt_v)
        zf16 = jnp.zeros((16,), jnp.float32)
        xlb = (xlb0_v, xlb1_v)
        gi = (gi0_v, gi1_v)
        sems = (sem0, sem1)
        iota = lax.iota(jnp.int32, 16)
        rotidx = []
        step = H
        while step < 16:
            rotidx.append(jnp.bitwise_and(iota + step, 15))
            step *= 2
        attc = [att_v[pl.ds(j * 16, 16)] for j in range(NCH)]

        def bucket_body(t, c):
            b = wid + t * NW

            @pl.when(b < B)
            def _():
                bstart = pl.multiple_of(pst_v[pl.ds(b, 16)][0], 16)
                bcnt = cnt_v[pl.ds(b, 16)][0]
                cpx = pltpu.async_copy(
                    xr_hbm.at[pl.ds(b * W_BKT * HC, W_BKT * HC)], xrf_v,
                    semx)

                def z1(i, c2):
                    acc_v[pl.ds(i * 16, 16)] = zf16
                    return c2
                lax.fori_loop(0, W_BKT * NCH, z1, 0)

                def z2(i, c2):
                    den_v[pl.ds(i * 16, 16)] = zf16
                    return c2
                lax.fori_loop(0, W_BKT, z2, 0)
                cpx.wait()

                def prefetch(q, ph):
                    for r in range(EBLK // 16):
                        sq = pl.ds(q * EBLK + r * 16, 16)
                        gi[ph][pl.ds(r * 16, 16)] = jnp.clip(
                            swin_v[sq], 0, NLIM - 1)
                    pltpu.async_copy(xl_hbm.at[gi[ph]], xlb[ph], sems[ph])

                def window(wi, c2):
                    wbase = bstart + wi * EWIN
                    pltpu.sync_copy(esrc_hbm.at[pl.ds(wbase, EWIN)],
                                    swin_v.at[pl.ds(0, EWIN)])
                    pltpu.sync_copy(edst_hbm.at[pl.ds(wbase, EWIN)],
                                    dwin_v.at[pl.ds(0, EWIN)])
                    nloc = jnp.minimum(bcnt - wi * EWIN, EWIN)
                    nblk = pl.cdiv(nloc, EBLK)
                    prefetch(0, 0)

                    def compute(q, ph):
                        pltpu.make_async_copy(xl_hbm.at[gi[ph]], xlb[ph],
                                              sems[ph]).wait()
                        nleft = jnp.minimum(nloc - q * EBLK, EBLK)
                        xlq = xlb[ph]

                        def logits(e, dstoff):
                            rbase = dstoff * HC
                            a0 = zf16
                            a1 = zf16
                            a2 = zf16
                            a3 = zf16
                            for j in range(NCH):
                                sl = pl.ds(j * 16, 16)
                                s = xlq[e, sl] + xrf_v[pl.ds(rbase + j * 16,
                                                             16)]
                                tlr = jnp.maximum(s, 0.2 * s)
                                prod = tlr * attc[j]
                                if j % 4 == 0:
                                    a0 = a0 + prod
                                elif j % 4 == 1:
                                    a1 = a1 + prod
                                elif j % 4 == 2:
                                    a2 = a2 + prod
                                else:
                                    a3 = a3 + prod
                            return (a0 + a1) + (a2 + a3)

                        def fold_exp(acc, rot_v):
                            for ridx in rotidx:
                                rot_v[...] = acc
                                acc = acc + plsc.load_gather(rot_v, [ridx])
                            return jnp.exp(acc)

                        def accumulate(e, dstoff, p):
                            rbase = dstoff * HC
                            den_v[pl.ds(dstoff * 16, 16)] += p
                            for j in range(NCH):
                                acc_v[pl.ds(rbase + j * 16, 16)] += (
                                    p * xlq[e, pl.ds(j * 16, 16)])

                        def dst_of(e):
                            return (dwin_v[pl.ds(q * EBLK + e, 16)][0]
                                    - b * W_BKT)

                        def pair(ii, c4):
                            e0 = ii * 2
                            e1 = e0 + 1
                            d0 = dst_of(e0)
                            d1 = dst_of(e1)
                            p0 = fold_exp(logits(e0, d0), rot0_v)
                            p1 = fold_exp(logits(e1, d1), rot1_v)
                            accumulate(e0, d0, p0)
                            accumulate(e1, d1, p1)
                            return c4
                        lax.fori_loop(0, nleft // 2, pair, 0)

                        @pl.when(nleft % 2 == 1)
                        def _():
                            e = nleft - 1
                            d = dst_of(e)
                            accumulate(e, d, fold_exp(logits(e, d), rot0_v))

                    def blkpair(qq, c3):
                        for ph in range(2):
                            q = qq * 2 + ph

                            @pl.when(q < nblk)
                            def _():
                                @pl.when(q + 1 < nblk)
                                def _():
                                    prefetch(q + 1, 1 - ph)
                                compute(q, ph)
                        return c3
                    lax.fori_loop(0, pl.cdiv(nblk, 2), blkpair, 0)
                    return c2
                lax.fori_loop(0, pl.cdiv(bcnt, EWIN), window, 0)
                pltpu.sync_copy(acc_v,
                                num_hbm.at[pl.ds(b * W_BKT * HC, W_BKT * HC)])
                pltpu.sync_copy(den_v,
                                den_hbm.at[pl.ds(b * W_BKT * 16, W_BKT * 16)])
            return c
        lax.fori_loop(0, nbk_max, bucket_body, 0)
    return k(xl, xr_flat, att_p, esrc, edst, pstart, cnt)


def _mm_pair_body(x_ref, wl_ref, wr_ref, ol_ref, or_ref):
    xb = x_ref[...]
    ol_ref[...] = jnp.dot(xb, wl_ref[...], preferred_element_type=jnp.float32)
    or_ref[...] = jnp.dot(xb, wr_ref[...], preferred_element_type=jnp.float32)


def _mm_pair(x, wl, wr):
    M, K = x.shape
    HC = wl.shape[1]
    BLK = 1024
    grid = (pl.cdiv(M, BLK),)
    return pl.pallas_call(
        _mm_pair_body,
        grid=grid,
        in_specs=[
            pl.BlockSpec((BLK, K), lambda i: (i, 0)),
            pl.BlockSpec((K, HC), lambda i: (0, 0)),
            pl.BlockSpec((K, HC), lambda i: (0, 0)),
        ],
        out_specs=[
            pl.BlockSpec((BLK, HC), lambda i: (i, 0)),
            pl.BlockSpec((BLK, HC), lambda i: (i, 0)),
        ],
        out_shape=[
            jax.ShapeDtypeStruct((M, HC), jnp.float32),
            jax.ShapeDtypeStruct((M, HC), jnp.float32),
        ],
    )(x, wl, wr)


def _epilogue_body(num_ref, den_ref, bias_ref, scale_ref, shift_ref, o_ref,
                   *, reps):
    num = num_ref[...]
    den = jnp.concatenate([den_ref[...]] * reps, axis=1)
    h = num / (den + 1e-16) + bias_ref[...]
    o_ref[...] = jnp.maximum(h * scale_ref[...] + shift_ref[...], 0.0)


def _epilogue(num, den, bias, bn_g, bn_b, bn_rm, bn_rv):
    n, HC = num.shape
    inv = bn_g / jnp.sqrt(bn_rv + 1e-5)
    shift = bn_b - bn_rm * inv
    BLK = 1024
    return pl.pallas_call(
        functools.partial(_epilogue_body, reps=HC // 16),
        grid=(pl.cdiv(n, BLK),),
        in_specs=[
            pl.BlockSpec((BLK, HC), lambda i: (i, 0)),
            pl.BlockSpec((BLK, 16), lambda i: (i, 0)),
            pl.BlockSpec((1, HC), lambda i: (0, 0)),
            pl.BlockSpec((1, HC), lambda i: (0, 0)),
            pl.BlockSpec((1, HC), lambda i: (0, 0)),
        ],
        out_specs=pl.BlockSpec((BLK, HC), lambda i: (i, 0)),
        out_shape=jax.ShapeDtypeStruct((n, HC), jnp.float32),
    )(num, den, bias.reshape(1, HC), inv.reshape(1, HC), shift.reshape(1, HC))


def _pool_body(b_ref, h_ref, sum_ref, cnt_ref, *, G):
    i = pl.program_id(0)
    bb = b_ref[0, 0, :]
    onehot = (lax.broadcasted_iota(jnp.int32, (G, bb.shape[0]), 0)
              == bb[None, :]).astype(jnp.float32)

    @pl.when(i == 0)
    def _():
        sum_ref[...] = jnp.zeros_like(sum_ref)
        cnt_ref[...] = jnp.zeros_like(cnt_ref)

    sum_ref[...] += jnp.dot(onehot, h_ref[...],
                            preferred_element_type=jnp.float32)
    cnt_ref[...] += jnp.dot(onehot,
                            jnp.ones((bb.shape[0], 128), jnp.float32),
                            preferred_element_type=jnp.float32)


def _pool(h, batch3, G):
    NP, HC = h.shape
    BLK = 256
    nblk = NP // BLK
    return pl.pallas_call(
        functools.partial(_pool_body, G=G),
        grid=(nblk,),
        in_specs=[
            pl.BlockSpec((1, 1, BLK), lambda i: (i, 0, 0)),
            pl.BlockSpec((BLK, HC), lambda i: (i, 0)),
        ],
        out_specs=[
            pl.BlockSpec((G, HC), lambda i: (0, 0)),
            pl.BlockSpec((G, 128), lambda i: (0, 0)),
        ],
        out_shape=[
            jax.ShapeDtypeStruct((G, HC), jnp.float32),
            jax.ShapeDtypeStruct((G, 128), jnp.float32),
        ],
    )(batch3, h)


def _head_body(sum_ref, cnt_ref, glob_ref, w1a_ref, w1b_ref, b1_ref,
               w2_ref, b2_ref, o_ref):
    cnt = jnp.maximum(cnt_ref[:, 0:1], 1.0)
    pooled = sum_ref[...] / cnt
    z = (jnp.dot(pooled, w1a_ref[...], preferred_element_type=jnp.float32)
         + jnp.dot(glob_ref[...], w1b_ref[...],
                   preferred_element_type=jnp.float32)
         + b1_ref[...])
    z = jnp.maximum(z, 0.0)
    o_ref[...] = jnp.dot(z, w2_ref[...],
                         preferred_element_type=jnp.float32) + b2_ref[...]


def _head(sums, cnts, globp, w1a, w1b, b1, w2, b2):
    G = sums.shape[0]
    return pl.pallas_call(
        _head_body,
        out_shape=jax.ShapeDtypeStruct((G, 8), jnp.float32),
    )(sums, cnts, globp, w1a, w1b, b1.reshape(1, -1), w2, b2.reshape(1, -1))


def kernel(x, edge_index, batch, global_feat, Wl1, Wr1, att1, bias1, bn1_g,
           bn1_b, bn1_rm, bn1_rv, Wl2, Wr2, att2, bias2, bn2_g, bn2_b,
           bn2_rm, bn2_rv, fc1_w, fc1_b, fc2_w, fc2_b):
    N = x.shape[0]
    E = edge_index.shape[1]
    G = global_feat.shape[0]
    H1, C1 = att1.shape
    H2, C2 = att2.shape
    HC1, HC2 = H1 * C1, H2 * C2

    B = -(-N // W_BKT)
    NB = _roundup(B + 1, 16)
    NB2 = NB + 16
    NPAD = B * W_BKT
    EPW = _roundup(-(-E // NW), BB)
    EPAD = NW * EPW + 16 * NB + EWIN

    p1 = jnp.asarray(_perm_idx(H1, C1))
    p2 = jnp.asarray(_perm_idx(H2, C2))

    Wl1p, Wr1p = Wl1[:, p1], Wr1[:, p1]
    att1p = att1.reshape(HC1)[p1]
    bias1p, g1p, b1p = bias1[p1], bn1_g[p1], bn1_b[p1]
    rm1p, rv1p = bn1_rm[p1], bn1_rv[p1]
    Wl2p, Wr2p = Wl2[p1][:, p2], Wr2[p1][:, p2]
    att2p = att2.reshape(HC2)[p2]
    bias2p, g2p, b2p = bias2[p2], bn2_g[p2], bn2_b[p2]
    rm2p, rv2p = bn2_rm[p2], bn2_rv[p2]
    fc1_wp = fc1_w[:HC2][p2]
    fc1_wg = jnp.pad(fc1_w[HC2:], ((0, 12), (0, 0)))
    globp = jnp.pad(global_feat, ((0, 0), (0, 12)))
    fc2_wp = jnp.pad(fc2_w, ((0, 0), (0, 7)))
    fc2_bp = jnp.pad(fc2_b, (0, 7))

    src_pad = jnp.pad(edge_index[0], (0, NW * EPW - E))
    dst_pad = jnp.pad(edge_index[1], (0, NW * EPW - E))

    hist = _bin_hist(dst_pad, E, NB, EPW)
    esrc, edst, pstart, cnt = _bin_place(src_pad, dst_pad, hist, E, NB,
                                         EPW, EPAD)

    xpad = jnp.pad(x, ((0, NPAD - N), (0, 0)))
    xl1, xr1 = _mm_pair(xpad, Wl1p, Wr1p)
    num1, den1 = _gat_edge(xl1, xr1.reshape(-1), att1p, esrc, edst, pstart,
                           cnt, H1, C1, B, NB2, N)
    h1 = _epilogue(num1.reshape(NPAD, HC1), den1.reshape(NPAD, 16),
                   bias1p, g1p, b1p, rm1p, rv1p)

    xl2, xr2 = _mm_pair(h1, Wl2p, Wr2p)
    num2, den2 = _gat_edge(xl2, xr2.reshape(-1), att2p, esrc, edst, pstart,
                           cnt, H2, C2, B, NB2, N)
    h2 = _epilogue(num2.reshape(NPAD, HC2), den2.reshape(NPAD, 16),
                   bias2p, g2p, b2p, rm2p, rv2p)

    NPOOL = _roundup(N, 256)
    hpool = jnp.pad(h2[:N], ((0, NPOOL - N), (0, 0)))
    bpool = jnp.pad(batch, (0, NPOOL - N), constant_values=G)
    batch3 = bpool.reshape(NPOOL // 256, 1, 256)
    sums, cnts = _pool(hpool, batch3, G)
    out = _head(sums, cnts, globp, fc1_wp, fc1_wg, fc1_b, fc2_wp, fc2_bp)
    return out[:, 0]


# speedup vs baseline: 17.2401x; 1.1366x over previous
import functools

import numpy as np
import jax
import jax.numpy as jnp
from jax import lax
from jax.experimental import pallas as pl
from jax.experimental.pallas import tpu as pltpu
from jax.experimental.pallas import tpu_sc as plsc

NW = 32
LBW = 6
W_BKT = 1 << LBW
BB = 2048
EBLK = 32
EWIN = 2048
RANK_BASE = 1


def _roundup(x, m):
    return (x + m - 1) // m * m


def _sc_mesh():
    return plsc.VectorSubcoreMesh(core_axis_name="c", subcore_axis_name="s")


_SC_PARAMS = pltpu.CompilerParams(needs_layout_passes=False)


def _perm_idx(H, C):
    HC = H * C
    k = np.arange(HC)
    j, l = k // 16, k % 16
    h = l % H
    c = j * (16 // H) + l // H
    return (h * C + c).astype(np.int32)


def _bin_hist(dst_pad, E, NB, EPW):
    nblk = EPW // BB

    @functools.partial(
        pl.kernel, mesh=_sc_mesh(), compiler_params=_SC_PARAMS,
        out_type=jax.ShapeDtypeStruct((NW, NB), jnp.int32),
        scratch_types=[
            pltpu.VMEM((BB,), jnp.int32),
            pltpu.VMEM((NB,), jnp.int32),
        ],
    )
    def k(dst_hbm, hist_hbm, buf_v, hist_v):
        wid = lax.axis_index("s") * 2 + lax.axis_index("c")
        wstart = wid * EPW
        zero16 = jnp.zeros((16,), jnp.int32)

        def zbody(i, c):
            hist_v[pl.ds(i * 16, 16)] = zero16
            return c
        lax.fori_loop(0, NB // 16, zbody, 0, unroll=True)

        iota = lax.iota(jnp.int32, 16)

        def blk(kb, c):
            pltpu.sync_copy(dst_hbm.at[pl.ds(wstart + kb * BB, BB)], buf_v)

            def chunk(i, c2):
                d = buf_v[pl.ds(i * 16, 16)]
                gidx = wstart + kb * BB + i * 16 + iota
                b = jnp.where(gidx < E, d >> LBW, NB - 1)
                rank, lastm = plsc.scan_count(b)
                plsc.addupdate_scatter(
                    hist_v, [b], rank + (1 - RANK_BASE), mask=lastm)
                return c2
            return lax.fori_loop(0, BB // 16, chunk, c)
        lax.fori_loop(0, nblk, blk, 0)
        pltpu.sync_copy(hist_v, hist_hbm.at[wid])
    return k(dst_pad)


def _bin_place(src_pad, dst_pad, hist, E, NB, EPW, EPAD):
    nblk = EPW // BB
    NB2 = NB + 16

    @functools.partial(
        pl.kernel, mesh=_sc_mesh(), compiler_params=_SC_PARAMS,
        out_type=(
            jax.ShapeDtypeStruct((EPAD,), jnp.int32),
            jax.ShapeDtypeStruct((EPAD,), jnp.int32),
            jax.ShapeDtypeStruct((NB2,), jnp.int32),
            jax.ShapeDtypeStruct((NB2,), jnp.int32),
        ),
        scratch_types=[
            pltpu.VMEM((NW, NB), jnp.int32),
            pltpu.VMEM((NB2,), jnp.int32),
            pltpu.VMEM((NB2,), jnp.int32),
            pltpu.VMEM((NB2,), jnp.int32),
            pltpu.VMEM((BB,), jnp.int32),
            pltpu.VMEM((BB,), jnp.int32),
            pltpu.VMEM((BB,), jnp.int32),
            pltpu.VMEM((BB,), jnp.int32),
            pltpu.VMEM((BB,), jnp.int32),
            pltpu.VMEM((BB,), jnp.int32),
            pltpu.SemaphoreType.DMA,
            pltpu.SemaphoreType.DMA,
            pltpu.SemaphoreType.DMA,
            pltpu.SemaphoreType.DMA,
        ],
    )
    def k(src_hbm, dst_hbm, hist_hbm, esrc_hbm, edst_hbm, pstart_hbm,
          cnt_hbm, histl_v, off_v, pst_v, tot_v, sbuf0_v, dbuf0_v, pos0_v,
          sbuf1_v, dbuf1_v, pos1_v, sem1, sem2, sem3, sem4):
        wid = lax.axis_index("s") * 2 + lax.axis_index("c")
        wstart = wid * EPW
        pltpu.sync_copy(hist_hbm, histl_v)
        zero16 = jnp.zeros((16,), jnp.int32)
        for cz in range(NB2 // 16):
            off_v[pl.ds(cz * 16, 16)] = zero16
            pst_v[pl.ds(cz * 16, 16)] = zero16
            tot_v[pl.ds(cz * 16, 16)] = zero16

        carry = jnp.int32(0)
        for c in range(NB // 16):
            sl = pl.ds(c * 16, 16)

            def addrow(wp, acc, _sl=sl):
                return acc + histl_v[wp, _sl]
            tot = lax.fori_loop(0, NW, addrow, zero16)
            mine = lax.fori_loop(0, wid, addrow, zero16)
            padded = jnp.bitwise_and(tot + 15, jnp.int32(-16))
            cums = plsc.cumsum(padded)
            pstart_c = cums - padded + carry
            off_v[sl] = pstart_c + mine
            pst_v[sl] = pstart_c
            tot_v[sl] = tot
            carry = carry + cums[15]

        @pl.when(wid == 0)
        def _():
            pltpu.sync_copy(pst_v, pstart_hbm)
            pltpu.sync_copy(tot_v, cnt_hbm)

        iota = lax.iota(jnp.int32, 16)
        sbuf = (sbuf0_v, sbuf1_v)
        dbuf = (dbuf0_v, dbuf1_v)
        pos = (pos0_v, pos1_v)
        ssem = (sem1, sem3)
        dsem = (sem2, sem4)
        pend = [None, None]
        for kb in range(nblk):
            ph = kb % 2
            if pend[ph] is not None:
                pend[ph][0].wait()
                pend[ph][1].wait()
            base = wstart + kb * BB
            pltpu.sync_copy(src_hbm.at[pl.ds(base, BB)], sbuf[ph])
            pltpu.sync_copy(dst_hbm.at[pl.ds(base, BB)], dbuf[ph])

            def chunk(i, c2, _base=base, _ph=ph):
                d = dbuf[_ph][pl.ds(i * 16, 16)]
                gidx = _base + i * 16 + iota
                b = jnp.where(gidx < E, d >> LBW, NB - 1)
                bs = plsc.load_gather(off_v, [b])
                rank, lastm = plsc.scan_count(b)
                p = jnp.minimum(bs + (rank - RANK_BASE), EPAD - 1)
                plsc.store_scatter(off_v, [b], p + 1, mask=lastm)
                pos[_ph][pl.ds(i * 16, 16)] = p
                return c2
            lax.fori_loop(0, BB // 16, chunk, 0)
            cp1 = pltpu.async_copy(sbuf[ph], esrc_hbm.at[pos[ph]], ssem[ph])
            cp2 = pltpu.async_copy(dbuf[ph], edst_hbm.at[pos[ph]], dsem[ph])
            pend[ph] = (cp1, cp2)
        for ph in range(2):
            if pend[ph] is not None:
                pend[ph][0].wait()
                pend[ph][1].wait()
    return k(src_pad, dst_pad, hist)


def _gat_edge(xl, xr_flat, att_p, esrc, edst, pstart, cnt, H, C, B, NB2,
              NLIM):
    HC = H * C
    NPAD = B * W_BKT
    NCH = HC // 16
    nbk_max = _roundup(B, NW) // NW

    @functools.partial(
        pl.kernel, mesh=_sc_mesh(), compiler_params=_SC_PARAMS,
        out_type=(
            jax.ShapeDtypeStruct((NPAD * HC,), jnp.float32),
            jax.ShapeDtypeStruct((NPAD * 16,), jnp.float32),
        ),
        scratch_types=[
            pltpu.VMEM((W_BKT * HC,), jnp.float32),
            pltpu.VMEM((W_BKT * HC,), jnp.float32),
            pltpu.VMEM((W_BKT * 16,), jnp.float32),
            pltpu.VMEM((EBLK, HC), jnp.float32),
            pltpu.VMEM((EBLK, HC), jnp.float32),
            pltpu.VMEM((EBLK,), jnp.int32),
            pltpu.VMEM((EBLK,), jnp.int32),
            pltpu.VMEM((HC,), jnp.float32),
            pltpu.VMEM((EWIN + 16,), jnp.int32),
            pltpu.VMEM((EWIN + 16,), jnp.int32),
            pltpu.VMEM((16,), jnp.float32),
            pltpu.VMEM((16,), jnp.float32),
            pltpu.VMEM((NB2,), jnp.int32),
            pltpu.VMEM((NB2,), jnp.int32),
            pltpu.SemaphoreType.DMA,
            pltpu.SemaphoreType.DMA,
            pltpu.SemaphoreType.DMA,
        ],
    )
    def k(xl_hbm, xr_hbm, att_hbm, esrc_hbm, edst_hbm, pstart_hbm, cnt_hbm,
          num_hbm, den_hbm, acc_v, xrf_v, den_v, xlb0_v, xlb1_v, gi0_v,
          gi1_v, att_v, swin_v, dwin_v, rot0_v, rot1_v, pst_v, cnt_v, semx,
          sem0, sem1):
        wid = lax.axis_index("s") * 2 + lax.axis_index("c")
        pltpu.sync_copy(att_hbm, att_v)
        pltpu.sync_copy(pstart_hbm, pst_v)
        pltpu.sync_copy(cnt_hbm, cnt_v)
        zf16 = jnp.zeros((16,), jnp.float32)
        xlb = (xlb0_v, xlb1_v)
        gi = (gi0_v, gi1_v)
        sems = (sem0, sem1)
        iota = lax.iota(jnp.int32, 16)
        rotidx = []
        step = H
        while step < 16:
            rotidx.append(jnp.bitwise_and(iota + step, 15))
            step *= 2
        attc = [att_v[pl.ds(j * 16, 16)] for j in range(NCH)]

        def bucket_body(t, c):
            b = wid + t * NW

            @pl.when(b < B)
            def _():
                bstart = pl.multiple_of(pst_v[pl.ds(b, 16)][0], 16)
                bcnt = cnt_v[pl.ds(b, 16)][0]
                cpx = pltpu.async_copy(
                    xr_hbm.at[pl.ds(b * W_BKT * HC, W_BKT * HC)], xrf_v,
                    semx)

                def z1(i, c2):
                    acc_v[pl.ds(i * 16, 16)] = zf16
                    return c2
                lax.fori_loop(0, W_BKT * NCH, z1, 0)

                def z2(i, c2):
                    den_v[pl.ds(i * 16, 16)] = zf16
                    return c2
                lax.fori_loop(0, W_BKT, z2, 0)
                cpx.wait()

                def prefetch(q, ph):
                    for r in range(EBLK // 16):
                        sq = pl.ds(q * EBLK + r * 16, 16)
                        gi[ph][pl.ds(r * 16, 16)] = jnp.clip(
                            swin_v[sq], 0, NLIM - 1)
                    pltpu.async_copy(xl_hbm.at[gi[ph]], xlb[ph], sems[ph])

                def window(wi, c2):
                    wbase = bstart + wi * EWIN
                    pltpu.sync_copy(esrc_hbm.at[pl.ds(wbase, EWIN)],
                                    swin_v.at[pl.ds(0, EWIN)])
                    pltpu.sync_copy(edst_hbm.at[pl.ds(wbase, EWIN)],
                                    dwin_v.at[pl.ds(0, EWIN)])
                    nloc = jnp.minimum(bcnt - wi * EWIN, EWIN)
                    nblk = pl.cdiv(nloc, EBLK)
                    prefetch(0, 0)

                    def compute(q, ph):
                        pltpu.make_async_copy(xl_hbm.at[gi[ph]], xlb[ph],
                                              sems[ph]).wait()
                        nleft = jnp.minimum(nloc - q * EBLK, EBLK)
                        xlq = xlb[ph]

                        def logits(e, dstoff):
                            rbase = dstoff * HC
                            a0 = zf16
                            a1 = zf16
                            a2 = zf16
                            a3 = zf16
                            for j in range(NCH):
                                sl = pl.ds(j * 16, 16)
                                s = xlq[e, sl] + xrf_v[pl.ds(rbase + j * 16,
                                                             16)]
                                tlr = jnp.maximum(s, 0.2 * s)
                                prod = tlr * attc[j]
                                if j % 4 == 0:
                                    a0 = a0 + prod
                                elif j % 4 == 1:
                                    a1 = a1 + prod
                                elif j % 4 == 2:
                                    a2 = a2 + prod
                                else:
                                    a3 = a3 + prod
                            return (a0 + a1) + (a2 + a3)

                        def fold_exp(acc, rot_v):
                            for ridx in rotidx:
                                rot_v[...] = acc
                                acc = acc + plsc.load_gather(rot_v, [ridx])
                            return jnp.exp(acc)

                        def accumulate(e, dstoff, p):
                            rbase = dstoff * HC
                            plsc.addupdate(den_v.at[pl.ds(dstoff * 16, 16)],
                                           p)
                            for j in range(NCH):
                                plsc.addupdate(
                                    acc_v.at[pl.ds(rbase + j * 16, 16)],
                                    p * xlq[e, pl.ds(j * 16, 16)])

                        def dst_of(e):
                            return (dwin_v[pl.ds(q * EBLK + e, 16)][0]
                                    - b * W_BKT)

                        def pair(ii, c4):
                            e0 = ii * 2
                            e1 = e0 + 1
                            d0 = dst_of(e0)
                            d1 = dst_of(e1)
                            p0 = fold_exp(logits(e0, d0), rot0_v)
                            p1 = fold_exp(logits(e1, d1), rot1_v)
                            accumulate(e0, d0, p0)
                            accumulate(e1, d1, p1)
                            return c4
                        lax.fori_loop(0, nleft // 2, pair, 0)

                        @pl.when(nleft % 2 == 1)
                        def _():
                            e = nleft - 1
                            d = dst_of(e)
                            accumulate(e, d, fold_exp(logits(e, d), rot0_v))

                    def blkpair(qq, c3):
                        for ph in range(2):
                            q = qq * 2 + ph

                            @pl.when(q < nblk)
                            def _():
                                @pl.when(q + 1 < nblk)
                                def _():
                                    prefetch(q + 1, 1 - ph)
                                compute(q, ph)
                        return c3
                    lax.fori_loop(0, pl.cdiv(nblk, 2), blkpair, 0)
                    return c2
                lax.fori_loop(0, pl.cdiv(bcnt, EWIN), window, 0)
                pltpu.sync_copy(acc_v,
                                num_hbm.at[pl.ds(b * W_BKT * HC, W_BKT * HC)])
                pltpu.sync_copy(den_v,
                                den_hbm.at[pl.ds(b * W_BKT * 16, W_BKT * 16)])
            return c
        lax.fori_loop(0, nbk_max, bucket_body, 0)
    return k(xl, xr_flat, att_p, esrc, edst, pstart, cnt)


def _mm_pair_body(x_ref, wl_ref, wr_ref, ol_ref, or_ref):
    xb = x_ref[...]
    ol_ref[...] = jnp.dot(xb, wl_ref[...], preferred_element_type=jnp.float32)
    or_ref[...] = jnp.dot(xb, wr_ref[...], preferred_element_type=jnp.float32)


def _mm_pair(x, wl, wr):
    M, K = x.shape
    HC = wl.shape[1]
    BLK = 1024
    grid = (pl.cdiv(M, BLK),)
    return pl.pallas_call(
        _mm_pair_body,
        grid=grid,
        in_specs=[
            pl.BlockSpec((BLK, K), lambda i: (i, 0)),
            pl.BlockSpec((K, HC), lambda i: (0, 0)),
            pl.BlockSpec((K, HC), lambda i: (0, 0)),
        ],
        out_specs=[
            pl.BlockSpec((BLK, HC), lambda i: (i, 0)),
            pl.BlockSpec((BLK, HC), lambda i: (i, 0)),
        ],
        out_shape=[
            jax.ShapeDtypeStruct((M, HC), jnp.float32),
            jax.ShapeDtypeStruct((M, HC), jnp.float32),
        ],
    )(x, wl, wr)


def _epilogue_body(num_ref, den_ref, bias_ref, scale_ref, shift_ref, o_ref,
                   *, reps):
    num = num_ref[...]
    den = jnp.concatenate([den_ref[...]] * reps, axis=1)
    h = num / (den + 1e-16) + bias_ref[...]
    o_ref[...] = jnp.maximum(h * scale_ref[...] + shift_ref[...], 0.0)


def _epilogue(num, den, bias, bn_g, bn_b, bn_rm, bn_rv):
    n, HC = num.shape
    inv = bn_g / jnp.sqrt(bn_rv + 1e-5)
    shift = bn_b - bn_rm * inv
    BLK = 1024
    return pl.pallas_call(
        functools.partial(_epilogue_body, reps=HC // 16),
        grid=(pl.cdiv(n, BLK),),
        in_specs=[
            pl.BlockSpec((BLK, HC), lambda i: (i, 0)),
            pl.BlockSpec((BLK, 16), lambda i: (i, 0)),
            pl.BlockSpec((1, HC), lambda i: (0, 0)),
            pl.BlockSpec((1, HC), lambda i: (0, 0)),
            pl.BlockSpec((1, HC), lambda i: (0, 0)),
        ],
        out_specs=pl.BlockSpec((BLK, HC), lambda i: (i, 0)),
        out_shape=jax.ShapeDtypeStruct((n, HC), jnp.float32),
    )(num, den, bias.reshape(1, HC), inv.reshape(1, HC), shift.reshape(1, HC))


def _pool_body(b_ref, h_ref, sum_ref, cnt_ref, *, G):
    i = pl.program_id(0)
    bb = b_ref[0, 0, :]
    onehot = (lax.broadcasted_iota(jnp.int32, (G, bb.shape[0]), 0)
              == bb[None, :]).astype(jnp.float32)

    @pl.when(i == 0)
    def _():
        sum_ref[...] = jnp.zeros_like(sum_ref)
        cnt_ref[...] = jnp.zeros_like(cnt_ref)

    sum_ref[...] += jnp.dot(onehot, h_ref[...],
                            preferred_element_type=jnp.float32)
    cnt_ref[...] += jnp.dot(onehot,
                            jnp.ones((bb.shape[0], 128), jnp.float32),
                            preferred_element_type=jnp.float32)


def _pool(h, batch3, G):
    NP, HC = h.shape
    BLK = 256
    nblk = NP // BLK
    return pl.pallas_call(
        functools.partial(_pool_body, G=G),
        grid=(nblk,),
        in_specs=[
            pl.BlockSpec((1, 1, BLK), lambda i: (i, 0, 0)),
            pl.BlockSpec((BLK, HC), lambda i: (i, 0)),
        ],
        out_specs=[
            pl.BlockSpec((G, HC), lambda i: (0, 0)),
            pl.BlockSpec((G, 128), lambda i: (0, 0)),
        ],
        out_shape=[
            jax.ShapeDtypeStruct((G, HC), jnp.float32),
            jax.ShapeDtypeStruct((G, 128), jnp.float32),
        ],
    )(batch3, h)


def _head_body(sum_ref, cnt_ref, glob_ref, w1a_ref, w1b_ref, b1_ref,
               w2_ref, b2_ref, o_ref):
    cnt = jnp.maximum(cnt_ref[:, 0:1], 1.0)
    pooled = sum_ref[...] / cnt
    z = (jnp.dot(pooled, w1a_ref[...], preferred_element_type=jnp.float32)
         + jnp.dot(glob_ref[...], w1b_ref[...],
                   preferred_element_type=jnp.float32)
         + b1_ref[...])
    z = jnp.maximum(z, 0.0)
    o_ref[...] = jnp.dot(z, w2_ref[...],
                         preferred_element_type=jnp.float32) + b2_ref[...]


def _head(sums, cnts, globp, w1a, w1b, b1, w2, b2):
    G = sums.shape[0]
    return pl.pallas_call(
        _head_body,
        out_shape=jax.ShapeDtypeStruct((G, 8), jnp.float32),
    )(sums, cnts, globp, w1a, w1b, b1.reshape(1, -1), w2, b2.reshape(1, -1))


def kernel(x, edge_index, batch, global_feat, Wl1, Wr1, att1, bias1, bn1_g,
           bn1_b, bn1_rm, bn1_rv, Wl2, Wr2, att2, bias2, bn2_g, bn2_b,
           bn2_rm, bn2_rv, fc1_w, fc1_b, fc2_w, fc2_b):
    N = x.shape[0]
    E = edge_index.shape[1]
    G = global_feat.shape[0]
    H1, C1 = att1.shape
    H2, C2 = att2.shape
    HC1, HC2 = H1 * C1, H2 * C2

    B = -(-N // W_BKT)
    NB = _roundup(B + 1, 16)
    NB2 = NB + 16
    NPAD = B * W_BKT
    EPW = _roundup(-(-E // NW), BB)
    EPAD = NW * EPW + 16 * NB + EWIN

    p1 = jnp.asarray(_perm_idx(H1, C1))
    p2 = jnp.asarray(_perm_idx(H2, C2))

    Wl1p, Wr1p = Wl1[:, p1], Wr1[:, p1]
    att1p = att1.reshape(HC1)[p1]
    bias1p, g1p, b1p = bias1[p1], bn1_g[p1], bn1_b[p1]
    rm1p, rv1p = bn1_rm[p1], bn1_rv[p1]
    Wl2p, Wr2p = Wl2[p1][:, p2], Wr2[p1][:, p2]
    att2p = att2.reshape(HC2)[p2]
    bias2p, g2p, b2p = bias2[p2], bn2_g[p2], bn2_b[p2]
    rm2p, rv2p = bn2_rm[p2], bn2_rv[p2]
    fc1_wp = fc1_w[:HC2][p2]
    fc1_wg = jnp.pad(fc1_w[HC2:], ((0, 12), (0, 0)))
    globp = jnp.pad(global_feat, ((0, 0), (0, 12)))
    fc2_wp = jnp.pad(fc2_w, ((0, 0), (0, 7)))
    fc2_bp = jnp.pad(fc2_b, (0, 7))

    src_pad = jnp.pad(edge_index[0], (0, NW * EPW - E))
    dst_pad = jnp.pad(edge_index[1], (0, NW * EPW - E))

    hist = _bin_hist(dst_pad, E, NB, EPW)
    esrc, edst, pstart, cnt = _bin_place(src_pad, dst_pad, hist, E, NB,
                                         EPW, EPAD)

    xpad = jnp.pad(x, ((0, NPAD - N), (0, 0)))
    xl1, xr1 = _mm_pair(xpad, Wl1p, Wr1p)
    num1, den1 = _gat_edge(xl1, xr1.reshape(-1), att1p, esrc, edst, pstart,
                           cnt, H1, C1, B, NB2, N)
    h1 = _epilogue(num1.reshape(NPAD, HC1), den1.reshape(NPAD, 16),
                   bias1p, g1p, b1p, rm1p, rv1p)

    xl2, xr2 = _mm_pair(h1, Wl2p, Wr2p)
    num2, den2 = _gat_edge(xl2, xr2.reshape(-1), att2p, esrc, edst, pstart,
                           cnt, H2, C2, B, NB2, N)
    h2 = _epilogue(num2.reshape(NPAD, HC2), den2.reshape(NPAD, 16),
                   bias2p, g2p, b2p, rm2p, rv2p)

    NPOOL = _roundup(N, 256)
    hpool = jnp.pad(h2[:N], ((0, NPOOL - N), (0, 0)))
    bpool = jnp.pad(batch, (0, NPOOL - N), constant_values=G)
    batch3 = bpool.reshape(NPOOL // 256, 1, 256)
    sums, cnts = _pool(hpool, batch3, G)
    out = _head(sums, cnts, globp, fc1_wp, fc1_wg, fc1_b, fc2_wp, fc2_bp)
    return out[:, 0]


# speedup vs baseline: 17.6415x; 1.0233x over previous
import functools

import numpy as np
import jax
import jax.numpy as jnp
from jax import lax
from jax.experimental import pallas as pl
from jax.experimental.pallas import tpu as pltpu
from jax.experimental.pallas import tpu_sc as plsc

NW = 32
LBW = 6
W_BKT = 1 << LBW
BB = 2048
EBLK = 32
EWIN = 2048
RANK_BASE = 1


def _roundup(x, m):
    return (x + m - 1) // m * m


def _sc_mesh():
    return plsc.VectorSubcoreMesh(core_axis_name="c", subcore_axis_name="s")


_SC_PARAMS = pltpu.CompilerParams(needs_layout_passes=False)


def _perm_idx(H, C):
    HC = H * C
    k = np.arange(HC)
    j, l = k // 16, k % 16
    h = l % H
    c = j * (16 // H) + l // H
    return (h * C + c).astype(np.int32)


def _bin_hist(dst_pad, E, NB, EPW):
    nblk = EPW // BB

    @functools.partial(
        pl.kernel, mesh=_sc_mesh(), compiler_params=_SC_PARAMS,
        out_type=jax.ShapeDtypeStruct((NW, NB), jnp.int32),
        scratch_types=[
            pltpu.VMEM((BB,), jnp.int32),
            pltpu.VMEM((NB,), jnp.int32),
        ],
    )
    def k(dst_hbm, hist_hbm, buf_v, hist_v):
        wid = lax.axis_index("s") * 2 + lax.axis_index("c")
        wstart = wid * EPW
        zero16 = jnp.zeros((16,), jnp.int32)

        def zbody(i, c):
            hist_v[pl.ds(i * 16, 16)] = zero16
            return c
        lax.fori_loop(0, NB // 16, zbody, 0, unroll=True)

        iota = lax.iota(jnp.int32, 16)

        def blk(kb, c):
            pltpu.sync_copy(dst_hbm.at[pl.ds(wstart + kb * BB, BB)], buf_v)

            def chunk(i, c2):
                d = buf_v[pl.ds(i * 16, 16)]
                gidx = wstart + kb * BB + i * 16 + iota
                b = jnp.where(gidx < E, d >> LBW, NB - 1)
                rank, lastm = plsc.scan_count(b)
                plsc.addupdate_scatter(
                    hist_v, [b], rank + (1 - RANK_BASE), mask=lastm)
                return c2
            return lax.fori_loop(0, BB // 16, chunk, c)
        lax.fori_loop(0, nblk, blk, 0)
        pltpu.sync_copy(hist_v, hist_hbm.at[wid])
    return k(dst_pad)


def _bin_place(src_pad, dst_pad, hist, E, NB, EPW, EPAD):
    nblk = EPW // BB
    NB2 = NB + 16

    @functools.partial(
        pl.kernel, mesh=_sc_mesh(), compiler_params=_SC_PARAMS,
        out_type=(
            jax.ShapeDtypeStruct((EPAD,), jnp.int32),
            jax.ShapeDtypeStruct((EPAD,), jnp.int32),
            jax.ShapeDtypeStruct((NB2,), jnp.int32),
            jax.ShapeDtypeStruct((NB2,), jnp.int32),
        ),
        scratch_types=[
            pltpu.VMEM((NW, NB), jnp.int32),
            pltpu.VMEM((NB2,), jnp.int32),
            pltpu.VMEM((NB2,), jnp.int32),
            pltpu.VMEM((NB2,), jnp.int32),
            pltpu.VMEM((BB,), jnp.int32),
            pltpu.VMEM((BB,), jnp.int32),
            pltpu.VMEM((BB,), jnp.int32),
            pltpu.VMEM((BB,), jnp.int32),
            pltpu.VMEM((BB,), jnp.int32),
            pltpu.VMEM((BB,), jnp.int32),
            pltpu.SemaphoreType.DMA,
            pltpu.SemaphoreType.DMA,
            pltpu.SemaphoreType.DMA,
            pltpu.SemaphoreType.DMA,
        ],
    )
    def k(src_hbm, dst_hbm, hist_hbm, esrc_hbm, edst_hbm, pstart_hbm,
          cnt_hbm, histl_v, off_v, pst_v, tot_v, sbuf0_v, dbuf0_v, pos0_v,
          sbuf1_v, dbuf1_v, pos1_v, sem1, sem2, sem3, sem4):
        wid = lax.axis_index("s") * 2 + lax.axis_index("c")
        wstart = wid * EPW
        pltpu.sync_copy(hist_hbm, histl_v)
        zero16 = jnp.zeros((16,), jnp.int32)
        for cz in range(NB2 // 16):
            off_v[pl.ds(cz * 16, 16)] = zero16
            pst_v[pl.ds(cz * 16, 16)] = zero16
            tot_v[pl.ds(cz * 16, 16)] = zero16

        carry = jnp.int32(0)
        for c in range(NB // 16):
            sl = pl.ds(c * 16, 16)

            def addrow(wp, acc, _sl=sl):
                return acc + histl_v[wp, _sl]
            tot = lax.fori_loop(0, NW, addrow, zero16)
            mine = lax.fori_loop(0, wid, addrow, zero16)
            padded = jnp.bitwise_and(tot + 15, jnp.int32(-16))
            cums = plsc.cumsum(padded)
            pstart_c = cums - padded + carry
            off_v[sl] = pstart_c + mine
            pst_v[sl] = pstart_c
            tot_v[sl] = tot
            carry = carry + cums[15]

        @pl.when(wid == 0)
        def _():
            pltpu.sync_copy(pst_v, pstart_hbm)
            pltpu.sync_copy(tot_v, cnt_hbm)

        iota = lax.iota(jnp.int32, 16)
        sbuf = (sbuf0_v, sbuf1_v)
        dbuf = (dbuf0_v, dbuf1_v)
        pos = (pos0_v, pos1_v)
        ssem = (sem1, sem3)
        dsem = (sem2, sem4)
        pend = [None, None]
        for kb in range(nblk):
            ph = kb % 2
            if pend[ph] is not None:
                pend[ph][0].wait()
                pend[ph][1].wait()
            base = wstart + kb * BB
            pltpu.sync_copy(src_hbm.at[pl.ds(base, BB)], sbuf[ph])
            pltpu.sync_copy(dst_hbm.at[pl.ds(base, BB)], dbuf[ph])

            def chunk(i, c2, _base=base, _ph=ph):
                d = dbuf[_ph][pl.ds(i * 16, 16)]
                gidx = _base + i * 16 + iota
                b = jnp.where(gidx < E, d >> LBW, NB - 1)
                bs = plsc.load_gather(off_v, [b])
                rank, lastm = plsc.scan_count(b)
                p = jnp.minimum(bs + (rank - RANK_BASE), EPAD - 1)
                plsc.store_scatter(off_v, [b], p + 1, mask=lastm)
                pos[_ph][pl.ds(i * 16, 16)] = p
                return c2
            lax.fori_loop(0, BB // 16, chunk, 0)
            cp1 = pltpu.async_copy(sbuf[ph], esrc_hbm.at[pos[ph]], ssem[ph])
            cp2 = pltpu.async_copy(dbuf[ph], edst_hbm.at[pos[ph]], dsem[ph])
            pend[ph] = (cp1, cp2)
        for ph in range(2):
            if pend[ph] is not None:
                pend[ph][0].wait()
                pend[ph][1].wait()
    return k(src_pad, dst_pad, hist)


def _gat_edge(xl, xr_flat, att_p, esrc, edst, pstart, cnt, H, C, B, NB2,
              NLIM):
    HC = H * C
    NPAD = B * W_BKT
    NCH = HC // 16
    nbk_max = _roundup(B, NW) // NW

    @functools.partial(
        pl.kernel, mesh=_sc_mesh(), compiler_params=_SC_PARAMS,
        out_type=(
            jax.ShapeDtypeStruct((NPAD * HC,), jnp.float32),
            jax.ShapeDtypeStruct((NPAD * 16,), jnp.float32),
        ),
        scratch_types=[
            pltpu.VMEM((W_BKT * HC,), jnp.float32),
            pltpu.VMEM((W_BKT * HC,), jnp.float32),
            pltpu.VMEM((W_BKT * 16,), jnp.float32),
            pltpu.VMEM((EBLK, HC), jnp.float32),
            pltpu.VMEM((EBLK, HC), jnp.float32),
            pltpu.VMEM((EBLK,), jnp.int32),
            pltpu.VMEM((EBLK,), jnp.int32),
            pltpu.VMEM((HC,), jnp.float32),
            pltpu.VMEM((EWIN + 16,), jnp.int32),
            pltpu.VMEM((EWIN + 16,), jnp.int32),
            pltpu.VMEM((16,), jnp.float32),
            pltpu.VMEM((16,), jnp.float32),
            pltpu.VMEM((NB2,), jnp.int32),
            pltpu.VMEM((NB2,), jnp.int32),
            pltpu.SemaphoreType.DMA,
            pltpu.SemaphoreType.DMA,
            pltpu.SemaphoreType.DMA,
        ],
    )
    def k(xl_hbm, xr_hbm, att_hbm, esrc_hbm, edst_hbm, pstart_hbm, cnt_hbm,
          num_hbm, den_hbm, acc_v, xrf_v, den_v, xlb0_v, xlb1_v, gi0_v,
          gi1_v, att_v, swin_v, dwin_v, rot0_v, rot1_v, pst_v, cnt_v, semx,
          sem0, sem1):
        wid = lax.axis_index("s") * 2 + lax.axis_index("c")
        pltpu.sync_copy(att_hbm, att_v)
        pltpu.sync_copy(pstart_hbm, pst_v)
        pltpu.sync_copy(cnt_hbm, cnt_v)
        zf16 = jnp.zeros((16,), jnp.float32)
        xlb = (xlb0_v, xlb1_v)
        gi = (gi0_v, gi1_v)
        sems = (sem0, sem1)
        iota = lax.iota(jnp.int32, 16)
        rotidx = []
        step = H
        while step < 16:
            rotidx.append(jnp.bitwise_and(iota + step, 15))
            step *= 2
        attc = [att_v[pl.ds(j * 16, 16)] for j in range(NCH)]

        def bucket_body(t, c):
            b = wid + t * NW

            @pl.when(b < B)
            def _():
                bstart = pl.multiple_of(pst_v[pl.ds(b, 16)][0], 16)
                bcnt = cnt_v[pl.ds(b, 16)][0]
                cpx = pltpu.async_copy(
                    xr_hbm.at[pl.ds(b * W_BKT * HC, W_BKT * HC)], xrf_v,
                    semx)

                def z1(i, c2):
                    for u in range(8):
                        acc_v[pl.ds((i * 8 + u) * 16, 16)] = zf16
                    return c2
                lax.fori_loop(0, W_BKT * NCH // 8, z1, 0)

                def z2(i, c2):
                    for u in range(4):
                        den_v[pl.ds((i * 4 + u) * 16, 16)] = zf16
                    return c2
                lax.fori_loop(0, W_BKT // 4, z2, 0)
                cpx.wait()

                def prefetch(q, ph):
                    for r in range(EBLK // 16):
                        sq = pl.ds(q * EBLK + r * 16, 16)
                        gi[ph][pl.ds(r * 16, 16)] = jnp.clip(
                            swin_v[sq], 0, NLIM - 1)
                    pltpu.async_copy(xl_hbm.at[gi[ph]], xlb[ph], sems[ph])

                def window(wi, c2):
                    wbase = bstart + wi * EWIN
                    pltpu.sync_copy(esrc_hbm.at[pl.ds(wbase, EWIN)],
                                    swin_v.at[pl.ds(0, EWIN)])
                    pltpu.sync_copy(edst_hbm.at[pl.ds(wbase, EWIN)],
                                    dwin_v.at[pl.ds(0, EWIN)])
                    nloc = jnp.minimum(bcnt - wi * EWIN, EWIN)
                    nblk = pl.cdiv(nloc, EBLK)
                    prefetch(0, 0)

                    def compute(q, ph):
                        pltpu.make_async_copy(xl_hbm.at[gi[ph]], xlb[ph],
                                              sems[ph]).wait()
                        nleft = jnp.minimum(nloc - q * EBLK, EBLK)
                        xlq = xlb[ph]

                        def logits(e, dstoff):
                            rbase = dstoff * HC
                            a0 = zf16
                            a1 = zf16
                            a2 = zf16
                            a3 = zf16
                            for j in range(NCH):
                                sl = pl.ds(j * 16, 16)
                                s = xlq[e, sl] + xrf_v[pl.ds(rbase + j * 16,
                                                             16)]
                                tlr = jnp.maximum(s, 0.2 * s)
                                prod = tlr * attc[j]
                                if j % 4 == 0:
                                    a0 = a0 + prod
                                elif j % 4 == 1:
                                    a1 = a1 + prod
                                elif j % 4 == 2:
                                    a2 = a2 + prod
                                else:
                                    a3 = a3 + prod
                            return (a0 + a1) + (a2 + a3)

                        def fold_exp(acc, rot_v):
                            for ridx in rotidx:
                                rot_v[...] = acc
                                acc = acc + plsc.load_gather(rot_v, [ridx])
                            return jnp.exp(acc)

                        def accumulate(e, dstoff, p):
                            rbase = dstoff * HC
                            plsc.addupdate(den_v.at[pl.ds(dstoff * 16, 16)],
                                           p)
                            for j in range(NCH):
                                plsc.addupdate(
                                    acc_v.at[pl.ds(rbase + j * 16, 16)],
                                    p * xlq[e, pl.ds(j * 16, 16)])

                        def dst_of(e):
                            return (dwin_v[pl.ds(q * EBLK + e, 16)][0]
                                    - b * W_BKT)

                        def pair(ii, c4):
                            e0 = ii * 2
                            e1 = e0 + 1
                            d0 = dst_of(e0)
                            d1 = dst_of(e1)
                            p0 = fold_exp(logits(e0, d0), rot0_v)
                            p1 = fold_exp(logits(e1, d1), rot1_v)
                            accumulate(e0, d0, p0)
                            accumulate(e1, d1, p1)
                            return c4
                        lax.fori_loop(0, nleft // 2, pair, 0)

                        @pl.when(nleft % 2 == 1)
                        def _():
                            e = nleft - 1
                            d = dst_of(e)
                            accumulate(e, d, fold_exp(logits(e, d), rot0_v))

                    def blkpair(qq, c3):
                        for ph in range(2):
                            q = qq * 2 + ph

                            @pl.when(q < nblk)
                            def _():
                                @pl.when(q + 1 < nblk)
                                def _():
                                    prefetch(q + 1, 1 - ph)
                                compute(q, ph)
                        return c3
                    lax.fori_loop(0, pl.cdiv(nblk, 2), blkpair, 0)
                    return c2
                lax.fori_loop(0, pl.cdiv(bcnt, EWIN), window, 0)
                pltpu.sync_copy(acc_v,
                                num_hbm.at[pl.ds(b * W_BKT * HC, W_BKT * HC)])
                pltpu.sync_copy(den_v,
                                den_hbm.at[pl.ds(b * W_BKT * 16, W_BKT * 16)])
            return c
        lax.fori_loop(0, nbk_max, bucket_body, 0)
    return k(xl, xr_flat, att_p, esrc, edst, pstart, cnt)


def _mm_pair_body(x_ref, wl_ref, wr_ref, ol_ref, or_ref):
    xb = x_ref[...]
    ol_ref[...] = jnp.dot(xb, wl_ref[...], preferred_element_type=jnp.float32)
    or_ref[...] = jnp.dot(xb, wr_ref[...], preferred_element_type=jnp.float32)


def _mm_pair(x, wl, wr):
    M, K = x.shape
    HC = wl.shape[1]
    BLK = 1024
    grid = (pl.cdiv(M, BLK),)
    return pl.pallas_call(
        _mm_pair_body,
        grid=grid,
        in_specs=[
            pl.BlockSpec((BLK, K), lambda i: (i, 0)),
            pl.BlockSpec((K, HC), lambda i: (0, 0)),
            pl.BlockSpec((K, HC), lambda i: (0, 0)),
        ],
        out_specs=[
            pl.BlockSpec((BLK, HC), lambda i: (i, 0)),
            pl.BlockSpec((BLK, HC), lambda i: (i, 0)),
        ],
        out_shape=[
            jax.ShapeDtypeStruct((M, HC), jnp.float32),
            jax.ShapeDtypeStruct((M, HC), jnp.float32),
        ],
    )(x, wl, wr)


def _epilogue_body(num_ref, den_ref, bias_ref, scale_ref, shift_ref, o_ref,
                   *, reps):
    num = num_ref[...]
    den = jnp.concatenate([den_ref[...]] * reps, axis=1)
    h = num / (den + 1e-16) + bias_ref[...]
    o_ref[...] = jnp.maximum(h * scale_ref[...] + shift_ref[...], 0.0)


def _epilogue(num, den, bias, bn_g, bn_b, bn_rm, bn_rv):
    n, HC = num.shape
    inv = bn_g / jnp.sqrt(bn_rv + 1e-5)
    shift = bn_b - bn_rm * inv
    BLK = 1024
    return pl.pallas_call(
        functools.partial(_epilogue_body, reps=HC // 16),
        grid=(pl.cdiv(n, BLK),),
        in_specs=[
            pl.BlockSpec((BLK, HC), lambda i: (i, 0)),
            pl.BlockSpec((BLK, 16), lambda i: (i, 0)),
            pl.BlockSpec((1, HC), lambda i: (0, 0)),
            pl.BlockSpec((1, HC), lambda i: (0, 0)),
            pl.BlockSpec((1, HC), lambda i: (0, 0)),
        ],
        out_specs=pl.BlockSpec((BLK, HC), lambda i: (i, 0)),
        out_shape=jax.ShapeDtypeStruct((n, HC), jnp.float32),
    )(num, den, bias.reshape(1, HC), inv.reshape(1, HC), shift.reshape(1, HC))


def _pool_body(b_ref, h_ref, sum_ref, cnt_ref, *, G):
    i = pl.program_id(0)
    bb = b_ref[0, 0, :]
    onehot = (lax.broadcasted_iota(jnp.int32, (G, bb.shape[0]), 0)
              == bb[None, :]).astype(jnp.float32)

    @pl.when(i == 0)
    def _():
        sum_ref[...] = jnp.zeros_like(sum_ref)
        cnt_ref[...] = jnp.zeros_like(cnt_ref)

    sum_ref[...] += jnp.dot(onehot, h_ref[...],
                            preferred_element_type=jnp.float32)
    cnt_ref[...] += jnp.dot(onehot,
                            jnp.ones((bb.shape[0], 128), jnp.float32),
                            preferred_element_type=jnp.float32)


def _pool(h, batch3, G):
    NP, HC = h.shape
    BLK = 256
    nblk = NP // BLK
    return pl.pallas_call(
        functools.partial(_pool_body, G=G),
        grid=(nblk,),
        in_specs=[
            pl.BlockSpec((1, 1, BLK), lambda i: (i, 0, 0)),
            pl.BlockSpec((BLK, HC), lambda i: (i, 0)),
        ],
        out_specs=[
            pl.BlockSpec((G, HC), lambda i: (0, 0)),
            pl.BlockSpec((G, 128), lambda i: (0, 0)),
        ],
        out_shape=[
            jax.ShapeDtypeStruct((G, HC), jnp.float32),
            jax.ShapeDtypeStruct((G, 128), jnp.float32),
        ],
    )(batch3, h)


def _head_body(sum_ref, cnt_ref, glob_ref, w1a_ref, w1b_ref, b1_ref,
               w2_ref, b2_ref, o_ref):
    cnt = jnp.maximum(cnt_ref[:, 0:1], 1.0)
    pooled = sum_ref[...] / cnt
    z = (jnp.dot(pooled, w1a_ref[...], preferred_element_type=jnp.float32)
         + jnp.dot(glob_ref[...], w1b_ref[...],
                   preferred_element_type=jnp.float32)
         + b1_ref[...])
    z = jnp.maximum(z, 0.0)
    o_ref[...] = jnp.dot(z, w2_ref[...],
                         preferred_element_type=jnp.float32) + b2_ref[...]


def _head(sums, cnts, globp, w1a, w1b, b1, w2, b2):
    G = sums.shape[0]
    return pl.pallas_call(
        _head_body,
        out_shape=jax.ShapeDtypeStruct((G, 8), jnp.float32),
    )(sums, cnts, globp, w1a, w1b, b1.reshape(1, -1), w2, b2.reshape(1, -1))


def kernel(x, edge_index, batch, global_feat, Wl1, Wr1, att1, bias1, bn1_g,
           bn1_b, bn1_rm, bn1_rv, Wl2, Wr2, att2, bias2, bn2_g, bn2_b,
           bn2_rm, bn2_rv, fc1_w, fc1_b, fc2_w, fc2_b):
    N = x.shape[0]
    E = edge_index.shape[1]
    G = global_feat.shape[0]
    H1, C1 = att1.shape
    H2, C2 = att2.shape
    HC1, HC2 = H1 * C1, H2 * C2

    B = -(-N // W_BKT)
    NB = _roundup(B + 1, 16)
    NB2 = NB + 16
    NPAD = B * W_BKT
    EPW = _roundup(-(-E // NW), BB)
    EPAD = NW * EPW + 16 * NB + EWIN

    p1 = jnp.asarray(_perm_idx(H1, C1))
    p2 = jnp.asarray(_perm_idx(H2, C2))

    Wl1p, Wr1p = Wl1[:, p1], Wr1[:, p1]
    att1p = att1.reshape(HC1)[p1]
    bias1p, g1p, b1p = bias1[p1], bn1_g[p1], bn1_b[p1]
    rm1p, rv1p = bn1_rm[p1], bn1_rv[p1]
    Wl2p, Wr2p = Wl2[p1][:, p2], Wr2[p1][:, p2]
    att2p = att2.reshape(HC2)[p2]
    bias2p, g2p, b2p = bias2[p2], bn2_g[p2], bn2_b[p2]
    rm2p, rv2p = bn2_rm[p2], bn2_rv[p2]
    fc1_wp = fc1_w[:HC2][p2]
    fc1_wg = jnp.pad(fc1_w[HC2:], ((0, 12), (0, 0)))
    globp = jnp.pad(global_feat, ((0, 0), (0, 12)))
    fc2_wp = jnp.pad(fc2_w, ((0, 0), (0, 7)))
    fc2_bp = jnp.pad(fc2_b, (0, 7))

    src_pad = jnp.pad(edge_index[0], (0, NW * EPW - E))
    dst_pad = jnp.pad(edge_index[1], (0, NW * EPW - E))

    hist = _bin_hist(dst_pad, E, NB, EPW)
    esrc, edst, pstart, cnt = _bin_place(src_pad, dst_pad, hist, E, NB,
                                         EPW, EPAD)

    xpad = jnp.pad(x, ((0, NPAD - N), (0, 0)))
    xl1, xr1 = _mm_pair(xpad, Wl1p, Wr1p)
    num1, den1 = _gat_edge(xl1, xr1.reshape(-1), att1p, esrc, edst, pstart,
                           cnt, H1, C1, B, NB2, N)
    h1 = _epilogue(num1.reshape(NPAD, HC1), den1.reshape(NPAD, 16),
                   bias1p, g1p, b1p, rm1p, rv1p)

    xl2, xr2 = _mm_pair(h1, Wl2p, Wr2p)
    num2, den2 = _gat_edge(xl2, xr2.reshape(-1), att2p, esrc, edst, pstart,
                           cnt, H2, C2, B, NB2, N)
    h2 = _epilogue(num2.reshape(NPAD, HC2), den2.reshape(NPAD, 16),
                   bias2p, g2p, b2p, rm2p, rv2p)

    NPOOL = _roundup(N, 256)
    hpool = jnp.pad(h2[:N], ((0, NPOOL - N), (0, 0)))
    bpool = jnp.pad(batch, (0, NPOOL - N), constant_values=G)
    batch3 = bpool.reshape(NPOOL // 256, 1, 256)
    sums, cnts = _pool(hpool, batch3, G)
    out = _head(sums, cnts, globp, fc1_wp, fc1_wg, fc1_b, fc2_wp, fc2_bp)
    return out[:, 0]
